# Initial kernel scaffold; baseline (speedup 1.0000x reference)
import jax
import jax.numpy as jnp
from jax.experimental import pallas as pl


def kernel(x, edges, t, senders, receivers, params):
    raise NotImplementedError("write your pallas kernel here")


# speedup vs baseline: 2.5660x; 2.5660x over previous
import functools

import numpy as np
import jax
import jax.numpy as jnp
from jax import lax
from jax.experimental import pallas as pl
from jax.experimental.pallas import tpu as pltpu
from jax.experimental.pallas import tpu_sc as plsc

_B = 8
_NPG = 1250
_EPG = 40000
_N = _B * _NPG
_E = _B * _EPG
_H = 128
_SIGMA = 25.0

_NC = 2
_NS = 16
_NW = _NC * _NS
_CH = 80
_EPW = _E // _NW
_NCHUNK = _EPW // _CH
_EPC = _E // _NC
_NPC = _N // _NC
_ZROWS = 312
_ZTAIL = _NPC - _NS * _ZROWS

_BE = 2000


def _temb_body(t_ref, gfp_ref, twa_ref, twb_ref, tb_ref, wsum_ref, wn1_ref,
               be_ref, bn_ref, eb1_ref, nb1_ref, temb2_ref, inv_ref):
    t = t_ref[...]
    tembs = []
    for l in range(3):
        proj = t * gfp_ref[l] * (2.0 * np.pi)
        temb = (jnp.sin(proj) @ twa_ref[l]
                + jnp.cos(proj) @ twb_ref[l]
                + tb_ref[l])
        tembs.append(temb)
    for l in range(3):
        if l == 0:
            eb1_ref[0] = jnp.broadcast_to(be_ref[0], (_B, _H))
            nb1_ref[0] = jnp.broadcast_to(bn_ref[0], (_B, _H))
        else:
            eb1_ref[l] = tembs[l - 1] @ wsum_ref[l] + be_ref[l]
            nb1_ref[l] = tembs[l - 1] @ wn1_ref[l] + bn_ref[l]
    temb2_ref[...] = tembs[2]
    lsig = float(np.log(_SIGMA))
    var = (jnp.exp((2.0 * lsig) * t) - 1.0) / (2.0 * lsig)
    inv_ref[...] = lax.rsqrt(jnp.broadcast_to(var, (_B, _H)))


def _proj_body(xs_ref, w_ref, out_ref):
    out_ref[0] = jnp.dot(xs_ref[...], w_ref[0],
                         preferred_element_type=jnp.float32)


def _edge_body(es_ref, g_ref, w1_ref, w2_ref, w3_ref, b2_ref, b3_ref,
               b1_ref, out_ref):
    h = jnp.dot(es_ref[...], w1_ref[...], preferred_element_type=jnp.float32)
    h = jnp.maximum(h + g_ref[...] + b1_ref[0], 0.0)
    h = jnp.dot(h, w2_ref[...], preferred_element_type=jnp.float32)
    h = jnp.maximum(h + b2_ref[...], 0.0)
    out_ref[...] = (jnp.dot(h, w3_ref[...], preferred_element_type=jnp.float32)
                    + b3_ref[...])


def _node_body(xs_ref, agg_ref, wa_ref, wb_ref, w2_ref, w3_ref, b2_ref,
               b3_ref, b1_ref, out_ref):
    h = (jnp.dot(xs_ref[0], wa_ref[...], preferred_element_type=jnp.float32)
         + jnp.dot(agg_ref[0], wb_ref[...], preferred_element_type=jnp.float32)
         + b1_ref[0])
    h = jnp.maximum(h, 0.0)
    h = jnp.dot(h, w2_ref[...], preferred_element_type=jnp.float32)
    h = jnp.maximum(h + b2_ref[...], 0.0)
    out_ref[0] = (jnp.dot(h, w3_ref[...], preferred_element_type=jnp.float32)
                  + b3_ref[...])


def _node_final_body(xs_ref, agg_ref, wa_ref, wb_ref, w2_ref, w3_ref, b2_ref,
                     b3_ref, b1_ref, t2_ref, inv_ref, out_ref):
    h = (jnp.dot(xs_ref[0], wa_ref[...], preferred_element_type=jnp.float32)
         + jnp.dot(agg_ref[0], wb_ref[...], preferred_element_type=jnp.float32)
         + b1_ref[0])
    h = jnp.maximum(h, 0.0)
    h = jnp.dot(h, w2_ref[...], preferred_element_type=jnp.float32)
    h = jnp.maximum(h + b2_ref[...], 0.0)
    o = (jnp.dot(h, w3_ref[...], preferred_element_type=jnp.float32)
         + b3_ref[...] + t2_ref[0])
    out_ref[0] = o * inv_ref[0]


def _gather_body(p2, snd, rcv, out, idx_s, idx_r, buf_a, buf_b,
                 sem_a, sem_b):
    c = lax.axis_index("c")
    s = lax.axis_index("s")
    base0 = (c * _NS + s) * _EPW

    def chunk(j, carry):
        base = base0 + j * _CH
        pltpu.sync_copy(snd.at[pl.ds(base, _CH)], idx_s)
        pltpu.sync_copy(rcv.at[pl.ds(base, _CH)], idx_r)

        def shift(k, carry2):
            sl = pl.ds(k * 16, 16)
            idx_r[sl] = idx_r[sl] + _N
            return carry2
        lax.fori_loop(0, _CH // 16, shift, 0)

        cp_a = pltpu.async_copy(p2.at[idx_s], buf_a, sem_a)
        cp_b = pltpu.async_copy(p2.at[idx_r], buf_b, sem_b)
        cp_a.wait()
        cp_b.wait()

        def addrow(i, carry2):
            for k in range(_H // 16):
                sl = pl.ds(k * 16, 16)
                buf_a[i, sl] = buf_a[i, sl] + buf_b[i, sl]
            return carry2
        lax.fori_loop(0, _CH, addrow, 0)

        pltpu.sync_copy(buf_a, out.at[pl.ds(base, _CH)])
        return carry
    lax.fori_loop(0, _NCHUNK, chunk, 0)


def _scatter_body(en, rcv, zro, agg, sh, idxb, datab):
    c = lax.axis_index("c")
    s = lax.axis_index("s")
    half = c * _NPC

    pltpu.sync_copy(zro, sh.at[pl.ds(half + s * _ZROWS, _ZROWS)])

    @pl.when(s == 0)
    def _zero_tail():
        pltpu.sync_copy(zro.at[pl.ds(0, _ZTAIL)],
                        sh.at[pl.ds(half + _NS * _ZROWS, _ZTAIL)])

    plsc.subcore_barrier()

    base0 = c * _EPC + s * _EPW

    def chunk(j, carry):
        base = base0 + j * _CH
        pltpu.sync_copy(rcv.at[pl.ds(base, _CH)], idxb)
        pltpu.sync_copy(en.at[pl.ds(base, _CH)], datab)
        pltpu.sync_copy(datab, sh.at[idxb], add=True)
        return carry
    lax.fori_loop(0, _NCHUNK, chunk, 0)

    plsc.subcore_barrier()

    rows = pl.ds(half + s * _ZROWS, _ZROWS)
    pltpu.sync_copy(sh.at[rows], agg.at[rows])

    @pl.when(s == 0)
    def _write_tail():
        tail = pl.ds(half + _NS * _ZROWS, _ZTAIL)
        pltpu.sync_copy(sh.at[tail], agg.at[tail])


@functools.cache
def _sc_kernels():
    mesh = plsc.VectorSubcoreMesh(core_axis_name="c", subcore_axis_name="s",
                                  num_cores=_NC, num_subcores=_NS)
    gather = pl.kernel(
        _gather_body,
        mesh=mesh,
        out_type=jax.ShapeDtypeStruct((_E, _H), jnp.float32),
        scratch_types=[
            pltpu.VMEM((_CH,), jnp.int32),
            pltpu.VMEM((_CH,), jnp.int32),
            pltpu.VMEM((_CH, _H), jnp.float32),
            pltpu.VMEM((_CH, _H), jnp.float32),
            pltpu.SemaphoreType.DMA,
            pltpu.SemaphoreType.DMA,
        ],
    )
    scatter = pl.kernel(
        _scatter_body,
        mesh=mesh,
        out_type=jax.ShapeDtypeStruct((_N, _H), jnp.float32),
        scratch_types=[
            pltpu.VMEM_SHARED((_N, _H), jnp.float32),
            pltpu.VMEM((_CH,), jnp.int32),
            pltpu.VMEM((_CH, _H), jnp.float32),
        ],
    )
    return gather, scatter


_f32 = jnp.float32


def _proj_call(xs, wsr):
    nb = _N // _BE
    return pl.pallas_call(
        _proj_body,
        grid=(2, nb),
        in_specs=[
            pl.BlockSpec((_BE, _H), lambda j, i: (i, 0)),
            pl.BlockSpec((1, _H, _H), lambda j, i: (j, 0, 0)),
        ],
        out_specs=pl.BlockSpec((1, _BE, _H), lambda j, i: (j, i, 0)),
        out_shape=jax.ShapeDtypeStruct((2, _N, _H), _f32),
    )(xs, wsr)


def _edge_call(es, g, w1, w2, w3, b2, b3, eb1):
    nb = _E // _BE
    bpg = _EPG // _BE
    return pl.pallas_call(
        _edge_body,
        grid=(nb,),
        in_specs=[
            pl.BlockSpec((_BE, _H), lambda i: (i, 0)),
            pl.BlockSpec((_BE, _H), lambda i: (i, 0)),
            pl.BlockSpec((_H, _H), lambda i: (0, 0)),
            pl.BlockSpec((_H, _H), lambda i: (0, 0)),
            pl.BlockSpec((_H, _H), lambda i: (0, 0)),
            pl.BlockSpec((1, _H), lambda i: (0, 0)),
            pl.BlockSpec((1, _H), lambda i: (0, 0)),
            pl.BlockSpec((1, 1, _H), lambda i: (i // bpg, 0, 0)),
        ],
        out_specs=pl.BlockSpec((_BE, _H), lambda i: (i, 0)),
        out_shape=jax.ShapeDtypeStruct((_E, _H), _f32),
    )(es, g, w1, w2, w3, b2, b3, eb1)


def _node_call(xs3, agg3, wa, wb, w2, w3, b2, b3, nb1):
    return pl.pallas_call(
        _node_body,
        grid=(_B,),
        in_specs=[
            pl.BlockSpec((1, _NPG, _H), lambda b: (b, 0, 0)),
            pl.BlockSpec((1, _NPG, _H), lambda b: (b, 0, 0)),
            pl.BlockSpec((_H, _H), lambda b: (0, 0)),
            pl.BlockSpec((_H, _H), lambda b: (0, 0)),
            pl.BlockSpec((_H, _H), lambda b: (0, 0)),
            pl.BlockSpec((_H, _H), lambda b: (0, 0)),
            pl.BlockSpec((1, _H), lambda b: (0, 0)),
            pl.BlockSpec((1, _H), lambda b: (0, 0)),
            pl.BlockSpec((1, 1, _H), lambda b: (b, 0, 0)),
        ],
        out_specs=pl.BlockSpec((1, _NPG, _H), lambda b: (b, 0, 0)),
        out_shape=jax.ShapeDtypeStruct((_B, _NPG, _H), _f32),
    )(xs3, agg3, wa, wb, w2, w3, b2, b3, nb1)


def _node_final_call(xs3, agg3, wa, wb, w2, w3, b2, b3, nb1, t2, inv):
    return pl.pallas_call(
        _node_final_body,
        grid=(_B,),
        in_specs=[
            pl.BlockSpec((1, _NPG, _H), lambda b: (b, 0, 0)),
            pl.BlockSpec((1, _NPG, _H), lambda b: (b, 0, 0)),
            pl.BlockSpec((_H, _H), lambda b: (0, 0)),
            pl.BlockSpec((_H, _H), lambda b: (0, 0)),
            pl.BlockSpec((_H, _H), lambda b: (0, 0)),
            pl.BlockSpec((_H, _H), lambda b: (0, 0)),
            pl.BlockSpec((1, _H), lambda b: (0, 0)),
            pl.BlockSpec((1, _H), lambda b: (0, 0)),
            pl.BlockSpec((1, 1, _H), lambda b: (b, 0, 0)),
            pl.BlockSpec((1, 1, _H), lambda b: (b, 0, 0)),
            pl.BlockSpec((1, 1, _H), lambda b: (b, 0, 0)),
        ],
        out_specs=pl.BlockSpec((1, _NPG, _H), lambda b: (b, 0, 0)),
        out_shape=jax.ShapeDtypeStruct((_B, _NPG, _H), _f32),
    )(xs3, agg3, wa, wb, w2, w3, b2, b3, nb1, t2, inv)


def _temb_call(t2d, gfp, twa, twb, tb, wsum, wn1, be, bn):
    return pl.pallas_call(
        _temb_body,
        out_shape=[
            jax.ShapeDtypeStruct((3, _B, _H), _f32),
            jax.ShapeDtypeStruct((3, _B, _H), _f32),
            jax.ShapeDtypeStruct((_B, _H), _f32),
            jax.ShapeDtypeStruct((_B, _H), _f32),
        ],
    )(t2d, gfp, twa, twb, tb, wsum, wn1, be, bn)


def kernel(x, edges, t, senders, receivers, params):
    lp = [params['layer%d' % l] for l in range(3)]
    w1e = [p['e_W'][0][0:_H] for p in lp]
    w1s = [p['e_W'][0][_H:2 * _H] for p in lp]
    w1r = [p['e_W'][0][2 * _H:3 * _H] for p in lp]
    wsum = jnp.stack([w1e[l] + w1s[l] + w1r[l] for l in range(3)])
    wsr = [jnp.stack([w1s[l], w1r[l]]) for l in range(3)]
    wn1a = [p['n_W'][0][0:_H] for p in lp]
    wn1b = [p['n_W'][0][_H:2 * _H] for p in lp]
    wn1 = jnp.stack(wn1a)
    gfp = jnp.stack([p['gfp_W'] for p in lp]).reshape(3, 1, _H // 2)
    twa = jnp.stack([p['t_W'][0:_H // 2] for p in lp])
    twb = jnp.stack([p['t_W'][_H // 2:] for p in lp])
    tb = jnp.stack([p['t_b'] for p in lp]).reshape(3, 1, _H)
    be = jnp.stack([p['e_b'][0] for p in lp]).reshape(3, 1, _H)
    bn = jnp.stack([p['n_b'][0] for p in lp]).reshape(3, 1, _H)
    t2d = t.reshape(_B, 1)

    eb1, nb1, temb2, inv = _temb_call(t2d, gfp, twa, twb, tb, wsum, wn1,
                                      be, bn)
    eb1_3 = [eb1[l].reshape(_B, 1, _H) for l in range(3)]
    nb1_3 = [nb1[l].reshape(_B, 1, _H) for l in range(3)]
    temb2_3 = temb2.reshape(_B, 1, _H)
    inv_3 = inv.reshape(_B, 1, _H)

    zro = jnp.zeros((_ZROWS, _H), _f32)
    senders = senders.astype(jnp.int32)
    receivers = receivers.astype(jnp.int32)

    gather_k, scatter_k = _sc_kernels()

    xs = x
    es = edges
    out3 = None
    for l in range(3):
        p = lp[l]
        p2 = _proj_call(xs, wsr[l]).reshape(2 * _N, _H)
        g = gather_k(p2, senders, receivers)
        en = _edge_call(es, g, w1e[l], p['e_W'][1], p['e_W'][2],
                        p['e_b'][1].reshape(1, _H), p['e_b'][2].reshape(1, _H),
                        eb1_3[l])
        agg = scatter_k(en, receivers, zro)
        xs3 = xs.reshape(_B, _NPG, _H)
        agg3 = agg.reshape(_B, _NPG, _H)
        if l < 2:
            xs = _node_call(xs3, agg3, wn1a[l], wn1b[l], p['n_W'][1],
                            p['n_W'][2], p['n_b'][1].reshape(1, _H),
                            p['n_b'][2].reshape(1, _H),
                            nb1_3[l]).reshape(_N, _H)
        else:
            out3 = _node_final_call(xs3, agg3, wn1a[l], wn1b[l], p['n_W'][1],
                                    p['n_W'][2], p['n_b'][1].reshape(1, _H),
                                    p['n_b'][2].reshape(1, _H),
                                    nb1_3[l], temb2_3, inv_3)
        es = en
    return out3.reshape(_B, _NPG * _H)


# speedup vs baseline: 3.5197x; 1.3717x over previous
import functools

import numpy as np
import jax
import jax.numpy as jnp
from jax import lax
from jax.experimental import pallas as pl
from jax.experimental.pallas import tpu as pltpu
from jax.experimental.pallas import tpu_sc as plsc

_B = 8
_NPG = 1250
_EPG = 40000
_N = _B * _NPG
_E = _B * _EPG
_H = 128
_SIGMA = 25.0

_NC = 2
_NS = 16
_NW = _NC * _NS
_CH = 80
_EPW = _E // _NW
_NCHUNK = _EPW // _CH
_EPC = _E // _NC
_NPC = _N // _NC
_ZROWS = 312
_ZTAIL = _NPC - _NS * _ZROWS

_BE = 2000


def _temb_body(t_ref, gfp_ref, twa_ref, twb_ref, tb_ref, wsum_ref, wn1_ref,
               be_ref, bn_ref, eb1_ref, nb1_ref, temb2_ref, inv_ref):
    t = t_ref[...]
    tembs = []
    for l in range(3):
        proj = t * gfp_ref[l] * (2.0 * np.pi)
        temb = (jnp.sin(proj) @ twa_ref[l]
                + jnp.cos(proj) @ twb_ref[l]
                + tb_ref[l])
        tembs.append(temb)
    for l in range(3):
        if l == 0:
            eb1_ref[0] = jnp.broadcast_to(be_ref[0], (_B, _H))
            nb1_ref[0] = jnp.broadcast_to(bn_ref[0], (_B, _H))
        else:
            eb1_ref[l] = tembs[l - 1] @ wsum_ref[l] + be_ref[l]
            nb1_ref[l] = tembs[l - 1] @ wn1_ref[l] + bn_ref[l]
    temb2_ref[...] = tembs[2]
    lsig = float(np.log(_SIGMA))
    var = (jnp.exp((2.0 * lsig) * t) - 1.0) / (2.0 * lsig)
    inv_ref[...] = lax.rsqrt(jnp.broadcast_to(var, (_B, _H)))


def _proj_body(xs_ref, ws_ref, wr_ref, outs_ref, outr_ref):
    x = xs_ref[...]
    outs_ref[...] = jnp.dot(x, ws_ref[...], preferred_element_type=jnp.float32)
    outr_ref[...] = jnp.dot(x, wr_ref[...], preferred_element_type=jnp.float32)


def _edge_body(es_ref, g_ref, w1_ref, w2_ref, w3_ref, b2_ref, b3_ref,
               b1_ref, out_ref):
    h = jnp.dot(es_ref[...], w1_ref[...], preferred_element_type=jnp.float32)
    h = jnp.maximum(h + g_ref[...] + b1_ref[0], 0.0)
    h = jnp.dot(h, w2_ref[...], preferred_element_type=jnp.float32)
    h = jnp.maximum(h + b2_ref[...], 0.0)
    out_ref[...] = (jnp.dot(h, w3_ref[...], preferred_element_type=jnp.float32)
                    + b3_ref[...])


def _node_body(xs_ref, agg_ref, wa_ref, wb_ref, w2_ref, w3_ref, b2_ref,
               b3_ref, b1_ref, out_ref):
    h = (jnp.dot(xs_ref[0], wa_ref[...], preferred_element_type=jnp.float32)
         + jnp.dot(agg_ref[0], wb_ref[...], preferred_element_type=jnp.float32)
         + b1_ref[0])
    h = jnp.maximum(h, 0.0)
    h = jnp.dot(h, w2_ref[...], preferred_element_type=jnp.float32)
    h = jnp.maximum(h + b2_ref[...], 0.0)
    out_ref[0] = (jnp.dot(h, w3_ref[...], preferred_element_type=jnp.float32)
                  + b3_ref[...])


def _node_final_body(xs_ref, agg_ref, wa_ref, wb_ref, w2_ref, w3_ref, b2_ref,
                     b3_ref, b1_ref, t2_ref, inv_ref, out_ref):
    h = (jnp.dot(xs_ref[0], wa_ref[...], preferred_element_type=jnp.float32)
         + jnp.dot(agg_ref[0], wb_ref[...], preferred_element_type=jnp.float32)
         + b1_ref[0])
    h = jnp.maximum(h, 0.0)
    h = jnp.dot(h, w2_ref[...], preferred_element_type=jnp.float32)
    h = jnp.maximum(h + b2_ref[...], 0.0)
    o = (jnp.dot(h, w3_ref[...], preferred_element_type=jnp.float32)
         + b3_ref[...] + t2_ref[0])
    out_ref[0] = o * inv_ref[0]


def _gather_body(ps, pr, snd, rcv, out,
                 idx_s0, idx_r0, idx_s1, idx_r1,
                 ba0, bb0, ba1, bb1,
                 sa0, sb0, sa1, sb1, sw0, sw1):
    c = lax.axis_index("c")
    s = lax.axis_index("s")
    base0 = (c * _NS + s) * _EPW
    slots = ((idx_s0, idx_r0, ba0, bb0, sa0, sb0, sw0),
             (idx_s1, idx_r1, ba1, bb1, sa1, sb1, sw1))

    def load_fire(j, slot, wait_write):
        idx_s, idx_r, ba, bb, sa, sb, sw = slot
        base = base0 + j * _CH
        pltpu.sync_copy(snd.at[pl.ds(base, _CH)], idx_s)
        pltpu.sync_copy(rcv.at[pl.ds(base, _CH)], idx_r)
        if wait_write:
            pltpu.make_async_copy(ba, out.at[pl.ds(base, _CH)], sw).wait()
        pltpu.async_copy(ps.at[idx_s], ba, sa)
        pltpu.async_copy(pr.at[idx_r], bb, sb)

    def finish(j, slot):
        idx_s, idx_r, ba, bb, sa, sb, sw = slot
        base = base0 + j * _CH
        pltpu.make_async_copy(ps.at[idx_s], ba, sa).wait()
        pltpu.make_async_copy(pr.at[idx_r], bb, sb).wait()

        def addrow(i, carry2):
            for k in range(_H // 16):
                sl = pl.ds(k * 16, 16)
                ba[i, sl] = ba[i, sl] + bb[i, sl]
            return carry2
        lax.fori_loop(0, _CH, addrow, 0)
        pltpu.async_copy(ba, out.at[pl.ds(base, _CH)], sw)

    load_fire(0, slots[0], False)
    load_fire(1, slots[1], False)

    def body(u, carry):
        j0 = 2 * u
        finish(j0, slots[0])
        load_fire(j0 + 2, slots[0], True)

        finish(j0 + 1, slots[1])

        @pl.when(j0 + 3 < _NCHUNK)
        def _refire1():
            load_fire(j0 + 3, slots[1], True)
        return carry
    lax.fori_loop(0, (_NCHUNK - 1) // 2, body, 0)

    finish(_NCHUNK - 1, slots[0])
    pltpu.make_async_copy(ba0, out.at[pl.ds(base0, _CH)], sw0).wait()
    pltpu.make_async_copy(ba1, out.at[pl.ds(base0, _CH)], sw1).wait()


def _scatter_body(en, rcv, zro, agg, sh, idx0, dat0, idx1, dat1, ss0, ss1):
    c = lax.axis_index("c")
    s = lax.axis_index("s")
    half = c * _NPC

    pltpu.sync_copy(zro, sh.at[pl.ds(half + s * _ZROWS, _ZROWS)])

    @pl.when(s == 0)
    def _zero_tail():
        pltpu.sync_copy(zro.at[pl.ds(0, _ZTAIL)],
                        sh.at[pl.ds(half + _NS * _ZROWS, _ZTAIL)])

    plsc.subcore_barrier()

    base0 = c * _EPC + s * _EPW
    slots = ((idx0, dat0, ss0), (idx1, dat1, ss1))

    def load_fire(j, slot):
        idx, dat, sem = slot
        base = base0 + j * _CH
        pltpu.sync_copy(rcv.at[pl.ds(base, _CH)], idx)
        pltpu.sync_copy(en.at[pl.ds(base, _CH)], dat)
        pltpu.async_copy(dat, sh.at[idx], sem, add=True)

    def wait_sc(slot):
        idx, dat, sem = slot
        pltpu.make_async_copy(dat, sh.at[idx], sem).wait()

    load_fire(0, slots[0])

    def body(u, carry):
        j0 = 2 * u
        load_fire(j0 + 1, slots[1])
        wait_sc(slots[0])
        load_fire(j0 + 2, slots[0])
        wait_sc(slots[1])
        return carry
    lax.fori_loop(0, (_NCHUNK - 1) // 2, body, 0)
    wait_sc(slots[0])

    plsc.subcore_barrier()

    rows = pl.ds(half + s * _ZROWS, _ZROWS)
    pltpu.sync_copy(sh.at[rows], agg.at[rows])

    @pl.when(s == 0)
    def _write_tail():
        tail = pl.ds(half + _NS * _ZROWS, _ZTAIL)
        pltpu.sync_copy(sh.at[tail], agg.at[tail])


@functools.cache
def _sc_kernels():
    mesh = plsc.VectorSubcoreMesh(core_axis_name="c", subcore_axis_name="s",
                                  num_cores=_NC, num_subcores=_NS)
    gather = pl.kernel(
        _gather_body,
        mesh=mesh,
        out_type=jax.ShapeDtypeStruct((_E, _H), jnp.float32),
        scratch_types=[
            pltpu.VMEM((_CH,), jnp.int32),
            pltpu.VMEM((_CH,), jnp.int32),
            pltpu.VMEM((_CH,), jnp.int32),
            pltpu.VMEM((_CH,), jnp.int32),
            pltpu.VMEM((_CH, _H), jnp.float32),
            pltpu.VMEM((_CH, _H), jnp.float32),
            pltpu.VMEM((_CH, _H), jnp.float32),
            pltpu.VMEM((_CH, _H), jnp.float32),
            pltpu.SemaphoreType.DMA,
            pltpu.SemaphoreType.DMA,
            pltpu.SemaphoreType.DMA,
            pltpu.SemaphoreType.DMA,
            pltpu.SemaphoreType.DMA,
            pltpu.SemaphoreType.DMA,
        ],
    )
    scatter = pl.kernel(
        _scatter_body,
        mesh=mesh,
        out_type=jax.ShapeDtypeStruct((_N, _H), jnp.float32),
        scratch_types=[
            pltpu.VMEM_SHARED((_N, _H), jnp.float32),
            pltpu.VMEM((_CH,), jnp.int32),
            pltpu.VMEM((_CH, _H), jnp.float32),
            pltpu.VMEM((_CH,), jnp.int32),
            pltpu.VMEM((_CH, _H), jnp.float32),
            pltpu.SemaphoreType.DMA,
            pltpu.SemaphoreType.DMA,
        ],
    )
    return gather, scatter


_f32 = jnp.float32


def _proj_call(xs, ws, wr):
    nb = _N // _BE
    return pl.pallas_call(
        _proj_body,
        grid=(nb,),
        in_specs=[
            pl.BlockSpec((_BE, _H), lambda i: (i, 0)),
            pl.BlockSpec((_H, _H), lambda i: (0, 0)),
            pl.BlockSpec((_H, _H), lambda i: (0, 0)),
        ],
        out_specs=[
            pl.BlockSpec((_BE, _H), lambda i: (i, 0)),
            pl.BlockSpec((_BE, _H), lambda i: (i, 0)),
        ],
        out_shape=[
            jax.ShapeDtypeStruct((_N, _H), _f32),
            jax.ShapeDtypeStruct((_N, _H), _f32),
        ],
    )(xs, ws, wr)


def _edge_call(es, g, w1, w2, w3, b2, b3, eb1):
    nb = _E // _BE
    bpg = _EPG // _BE
    return pl.pallas_call(
        _edge_body,
        grid=(nb,),
        in_specs=[
            pl.BlockSpec((_BE, _H), lambda i: (i, 0)),
            pl.BlockSpec((_BE, _H), lambda i: (i, 0)),
            pl.BlockSpec((_H, _H), lambda i: (0, 0)),
            pl.BlockSpec((_H, _H), lambda i: (0, 0)),
            pl.BlockSpec((_H, _H), lambda i: (0, 0)),
            pl.BlockSpec((1, _H), lambda i: (0, 0)),
            pl.BlockSpec((1, _H), lambda i: (0, 0)),
            pl.BlockSpec((1, 1, _H), lambda i: (i // bpg, 0, 0)),
        ],
        out_specs=pl.BlockSpec((_BE, _H), lambda i: (i, 0)),
        out_shape=jax.ShapeDtypeStruct((_E, _H), _f32),
    )(es, g, w1, w2, w3, b2, b3, eb1)


def _node_call(xs3, agg3, wa, wb, w2, w3, b2, b3, nb1):
    return pl.pallas_call(
        _node_body,
        grid=(_B,),
        in_specs=[
            pl.BlockSpec((1, _NPG, _H), lambda b: (b, 0, 0)),
            pl.BlockSpec((1, _NPG, _H), lambda b: (b, 0, 0)),
            pl.BlockSpec((_H, _H), lambda b: (0, 0)),
            pl.BlockSpec((_H, _H), lambda b: (0, 0)),
            pl.BlockSpec((_H, _H), lambda b: (0, 0)),
            pl.BlockSpec((_H, _H), lambda b: (0, 0)),
            pl.BlockSpec((1, _H), lambda b: (0, 0)),
            pl.BlockSpec((1, _H), lambda b: (0, 0)),
            pl.BlockSpec((1, 1, _H), lambda b: (b, 0, 0)),
        ],
        out_specs=pl.BlockSpec((1, _NPG, _H), lambda b: (b, 0, 0)),
        out_shape=jax.ShapeDtypeStruct((_B, _NPG, _H), _f32),
    )(xs3, agg3, wa, wb, w2, w3, b2, b3, nb1)


def _node_final_call(xs3, agg3, wa, wb, w2, w3, b2, b3, nb1, t2, inv):
    return pl.pallas_call(
        _node_final_body,
        grid=(_B,),
        in_specs=[
            pl.BlockSpec((1, _NPG, _H), lambda b: (b, 0, 0)),
            pl.BlockSpec((1, _NPG, _H), lambda b: (b, 0, 0)),
            pl.BlockSpec((_H, _H), lambda b: (0, 0)),
            pl.BlockSpec((_H, _H), lambda b: (0, 0)),
            pl.BlockSpec((_H, _H), lambda b: (0, 0)),
            pl.BlockSpec((_H, _H), lambda b: (0, 0)),
            pl.BlockSpec((1, _H), lambda b: (0, 0)),
            pl.BlockSpec((1, _H), lambda b: (0, 0)),
            pl.BlockSpec((1, 1, _H), lambda b: (b, 0, 0)),
            pl.BlockSpec((1, 1, _H), lambda b: (b, 0, 0)),
            pl.BlockSpec((1, 1, _H), lambda b: (b, 0, 0)),
        ],
        out_specs=pl.BlockSpec((1, _NPG, _H), lambda b: (b, 0, 0)),
        out_shape=jax.ShapeDtypeStruct((_B, _NPG, _H), _f32),
    )(xs3, agg3, wa, wb, w2, w3, b2, b3, nb1, t2, inv)


def _temb_call(t2d, gfp, twa, twb, tb, wsum, wn1, be, bn):
    return pl.pallas_call(
        _temb_body,
        out_shape=[
            jax.ShapeDtypeStruct((3, _B, _H), _f32),
            jax.ShapeDtypeStruct((3, _B, _H), _f32),
            jax.ShapeDtypeStruct((_B, _H), _f32),
            jax.ShapeDtypeStruct((_B, _H), _f32),
        ],
    )(t2d, gfp, twa, twb, tb, wsum, wn1, be, bn)


def kernel(x, edges, t, senders, receivers, params):
    lp = [params['layer%d' % l] for l in range(3)]
    w1e = [p['e_W'][0][0:_H] for p in lp]
    w1s = [p['e_W'][0][_H:2 * _H] for p in lp]
    w1r = [p['e_W'][0][2 * _H:3 * _H] for p in lp]
    wsum = jnp.stack([w1e[l] + w1s[l] + w1r[l] for l in range(3)])
    wn1a = [p['n_W'][0][0:_H] for p in lp]
    wn1b = [p['n_W'][0][_H:2 * _H] for p in lp]
    wn1 = jnp.stack(wn1a)
    gfp = jnp.stack([p['gfp_W'] for p in lp]).reshape(3, 1, _H // 2)
    twa = jnp.stack([p['t_W'][0:_H // 2] for p in lp])
    twb = jnp.stack([p['t_W'][_H // 2:] for p in lp])
    tb = jnp.stack([p['t_b'] for p in lp]).reshape(3, 1, _H)
    be = jnp.stack([p['e_b'][0] for p in lp]).reshape(3, 1, _H)
    bn = jnp.stack([p['n_b'][0] for p in lp]).reshape(3, 1, _H)
    t2d = t.reshape(_B, 1)

    eb1, nb1, temb2, inv = _temb_call(t2d, gfp, twa, twb, tb, wsum, wn1,
                                      be, bn)
    eb1_3 = [eb1[l].reshape(_B, 1, _H) for l in range(3)]
    nb1_3 = [nb1[l].reshape(_B, 1, _H) for l in range(3)]
    temb2_3 = temb2.reshape(_B, 1, _H)
    inv_3 = inv.reshape(_B, 1, _H)

    zro = jnp.zeros((_ZROWS, _H), _f32)
    senders = senders.astype(jnp.int32)
    receivers = receivers.astype(jnp.int32)

    gather_k, scatter_k = _sc_kernels()

    xs = x
    es = edges
    out3 = None
    for l in range(3):
        p = lp[l]
        p_s, p_r = _proj_call(xs, w1s[l], w1r[l])
        g = gather_k(p_s, p_r, senders, receivers)
        en = _edge_call(es, g, w1e[l], p['e_W'][1], p['e_W'][2],
                        p['e_b'][1].reshape(1, _H), p['e_b'][2].reshape(1, _H),
                        eb1_3[l])
        agg = scatter_k(en, receivers, zro)
        xs3 = xs.reshape(_B, _NPG, _H)
        agg3 = agg.reshape(_B, _NPG, _H)
        if l < 2:
            xs = _node_call(xs3, agg3, wn1a[l], wn1b[l], p['n_W'][1],
                            p['n_W'][2], p['n_b'][1].reshape(1, _H),
                            p['n_b'][2].reshape(1, _H),
                            nb1_3[l]).reshape(_N, _H)
        else:
            out3 = _node_final_call(xs3, agg3, wn1a[l], wn1b[l], p['n_W'][1],
                                    p['n_W'][2], p['n_b'][1].reshape(1, _H),
                                    p['n_b'][2].reshape(1, _H),
                                    nb1_3[l], temb2_3, inv_3)
        es = en
    return out3.reshape(_B, _NPG * _H)


# speedup vs baseline: 4.1385x; 1.1758x over previous
import functools

import numpy as np
import jax
import jax.numpy as jnp
from jax import lax
from jax.experimental import pallas as pl
from jax.experimental.pallas import tpu as pltpu
from jax.experimental.pallas import tpu_sc as plsc

_B = 8
_NPG = 1250
_EPG = 40000
_N = _B * _NPG
_E = _B * _EPG
_H = 128
_SIGMA = 25.0

_NC = 2
_NS = 16
_NW = _NC * _NS
_CH = 40
_MSUB = 5
_SUP = _CH * _MSUB
_EPW = _E // _NW
_NSUP = _EPW // _SUP
_EPC = _E // _NC
_NPC = _N // _NC
_ZROWS = 312
_ZTAIL = _NPC - _NS * _ZROWS

_BE = 2000


def _temb_body(t_ref, gfp_ref, twa_ref, twb_ref, tb_ref, wsum_ref, wn1_ref,
               be_ref, bn_ref, eb1_ref, nb1_ref, temb2_ref, inv_ref):
    t = t_ref[...]
    tembs = []
    for l in range(3):
        proj = t * gfp_ref[l] * (2.0 * np.pi)
        temb = (jnp.sin(proj) @ twa_ref[l]
                + jnp.cos(proj) @ twb_ref[l]
                + tb_ref[l])
        tembs.append(temb)
    for l in range(3):
        if l == 0:
            eb1_ref[0] = jnp.broadcast_to(be_ref[0], (_B, _H))
            nb1_ref[0] = jnp.broadcast_to(bn_ref[0], (_B, _H))
        else:
            eb1_ref[l] = tembs[l - 1] @ wsum_ref[l] + be_ref[l]
            nb1_ref[l] = tembs[l - 1] @ wn1_ref[l] + bn_ref[l]
    temb2_ref[...] = tembs[2]
    lsig = float(np.log(_SIGMA))
    var = (jnp.exp((2.0 * lsig) * t) - 1.0) / (2.0 * lsig)
    inv_ref[...] = lax.rsqrt(jnp.broadcast_to(var, (_B, _H)))


def _proj_body(xs_ref, ws_ref, wr_ref, outs_ref, outr_ref):
    x = xs_ref[...]
    outs_ref[...] = jnp.dot(x, ws_ref[...], preferred_element_type=jnp.float32)
    outr_ref[...] = jnp.dot(x, wr_ref[...], preferred_element_type=jnp.float32)


def _edge_body(es_ref, g_ref, w1_ref, w2_ref, w3_ref, b2_ref, b3_ref,
               b1_ref, out_ref):
    h = jnp.dot(es_ref[...], w1_ref[...], preferred_element_type=jnp.float32)
    h = jnp.maximum(h + g_ref[...] + b1_ref[0], 0.0)
    h = jnp.dot(h, w2_ref[...], preferred_element_type=jnp.float32)
    h = jnp.maximum(h + b2_ref[...], 0.0)
    out_ref[...] = (jnp.dot(h, w3_ref[...], preferred_element_type=jnp.float32)
                    + b3_ref[...])


def _node_body(xs_ref, agg_ref, wa_ref, wb_ref, w2_ref, w3_ref, b2_ref,
               b3_ref, b1_ref, out_ref):
    h = (jnp.dot(xs_ref[0], wa_ref[...], preferred_element_type=jnp.float32)
         + jnp.dot(agg_ref[0], wb_ref[...], preferred_element_type=jnp.float32)
         + b1_ref[0])
    h = jnp.maximum(h, 0.0)
    h = jnp.dot(h, w2_ref[...], preferred_element_type=jnp.float32)
    h = jnp.maximum(h + b2_ref[...], 0.0)
    out_ref[0] = (jnp.dot(h, w3_ref[...], preferred_element_type=jnp.float32)
                  + b3_ref[...])


def _node_final_body(xs_ref, agg_ref, wa_ref, wb_ref, w2_ref, w3_ref, b2_ref,
                     b3_ref, b1_ref, t2_ref, inv_ref, out_ref):
    h = (jnp.dot(xs_ref[0], wa_ref[...], preferred_element_type=jnp.float32)
         + jnp.dot(agg_ref[0], wb_ref[...], preferred_element_type=jnp.float32)
         + b1_ref[0])
    h = jnp.maximum(h, 0.0)
    h = jnp.dot(h, w2_ref[...], preferred_element_type=jnp.float32)
    h = jnp.maximum(h + b2_ref[...], 0.0)
    o = (jnp.dot(h, w3_ref[...], preferred_element_type=jnp.float32)
         + b3_ref[...] + t2_ref[0])
    out_ref[0] = o * inv_ref[0]


def _gather_body(ps, pr, snd3, rcv3, out,
                 idx_s0, idx_r0, idx_s1, idx_r1,
                 ba0, bb0, ba1, bb1,
                 sa0, sb0, sa1, sb1, sw0, sw1):
    c = lax.axis_index("c")
    s = lax.axis_index("s")
    q0 = (c * _NS + s) * _NSUP
    base0 = q0 * _SUP
    slots = ((idx_s0, idx_r0, ba0, bb0, sa0, sb0, sw0),
             (idx_s1, idx_r1, ba1, bb1, sa1, sb1, sw1))

    def load_fire(u, slot, wait_write):
        idx_s, idx_r, ba, bb, sa, sb, sw = slot
        pltpu.sync_copy(snd3.at[q0 + u], idx_s)
        pltpu.sync_copy(rcv3.at[q0 + u], idx_r)
        if wait_write:
            pltpu.make_async_copy(ba, out.at[pl.ds(0, _SUP)], sw).wait()
        for m in range(_MSUB):
            dst = pl.ds(m * _CH, _CH)
            pltpu.async_copy(ps.at[idx_s.at[m]], ba.at[dst], sa)
            pltpu.async_copy(pr.at[idx_r.at[m]], bb.at[dst], sb)

    def finish(u, slot):
        idx_s, idx_r, ba, bb, sa, sb, sw = slot
        base = base0 + u * _SUP
        for m in range(_MSUB):
            dst = pl.ds(m * _CH, _CH)
            pltpu.make_async_copy(ps.at[idx_s.at[m]], ba.at[dst], sa).wait()
            pltpu.make_async_copy(pr.at[idx_r.at[m]], bb.at[dst], sb).wait()

        def addrow(i, carry2):
            for k in range(_H // 16):
                sl = pl.ds(k * 16, 16)
                ba[i, sl] = ba[i, sl] + bb[i, sl]
            return carry2
        lax.fori_loop(0, _SUP, addrow, 0)
        pltpu.async_copy(ba, out.at[pl.ds(base, _SUP)], sw)

    load_fire(0, slots[0], False)
    load_fire(1, slots[1], False)

    def body(t, carry):
        u0 = 2 * t
        finish(u0, slots[0])

        @pl.when(u0 + 2 < _NSUP)
        def _refire0():
            load_fire(u0 + 2, slots[0], True)

        finish(u0 + 1, slots[1])

        @pl.when(u0 + 3 < _NSUP)
        def _refire1():
            load_fire(u0 + 3, slots[1], True)
        return carry
    lax.fori_loop(0, _NSUP // 2, body, 0)

    pltpu.make_async_copy(ba0, out.at[pl.ds(0, _SUP)], sw0).wait()
    pltpu.make_async_copy(ba1, out.at[pl.ds(0, _SUP)], sw1).wait()


def _scatter_body(en, rcv3l, zro, agg, sh, idx0, dat0, idx1, dat1, ss0, ss1):
    c = lax.axis_index("c")
    s = lax.axis_index("s")
    half = c * _NPC

    pltpu.sync_copy(zro, sh.at[pl.ds(s * _ZROWS, _ZROWS)])

    @pl.when(s == 0)
    def _zero_tail():
        pltpu.sync_copy(zro.at[pl.ds(0, _ZTAIL)],
                        sh.at[pl.ds(_NS * _ZROWS, _ZTAIL)])

    plsc.subcore_barrier()

    q0 = (c * _NS + s) * _NSUP
    base0 = q0 * _SUP
    slots = ((idx0, dat0, ss0), (idx1, dat1, ss1))

    def load_fire(u, slot):
        idx, dat, sem = slot
        base = base0 + u * _SUP
        pltpu.sync_copy(rcv3l.at[q0 + u], idx)
        pltpu.sync_copy(en.at[pl.ds(base, _SUP)], dat)
        for m in range(_MSUB):
            src = pl.ds(m * _CH, _CH)
            pltpu.async_copy(dat.at[src], sh.at[idx.at[m]], sem, add=True)

    def wait_sc(slot):
        idx, dat, sem = slot
        for m in range(_MSUB):
            src = pl.ds(m * _CH, _CH)
            pltpu.make_async_copy(dat.at[src], sh.at[idx.at[m]], sem).wait()

    load_fire(0, slots[0])

    def body(t, carry):
        u0 = 2 * t
        load_fire(u0 + 1, slots[1])
        wait_sc(slots[0])

        @pl.when(u0 + 2 < _NSUP)
        def _refire0():
            load_fire(u0 + 2, slots[0])
        wait_sc(slots[1])
        return carry
    lax.fori_loop(0, _NSUP // 2, body, 0)

    plsc.subcore_barrier()

    pltpu.sync_copy(sh.at[pl.ds(s * _ZROWS, _ZROWS)],
                    agg.at[pl.ds(half + s * _ZROWS, _ZROWS)])

    @pl.when(s == 0)
    def _write_tail():
        pltpu.sync_copy(sh.at[pl.ds(_NS * _ZROWS, _ZTAIL)],
                        agg.at[pl.ds(half + _NS * _ZROWS, _ZTAIL)])


@functools.cache
def _sc_kernels():
    mesh = plsc.VectorSubcoreMesh(core_axis_name="c", subcore_axis_name="s",
                                  num_cores=_NC, num_subcores=_NS)
    gather = pl.kernel(
        _gather_body,
        mesh=mesh,
        out_type=jax.ShapeDtypeStruct((_E, _H), jnp.float32),
        scratch_types=[
            pltpu.VMEM((_MSUB, _CH), jnp.int32),
            pltpu.VMEM((_MSUB, _CH), jnp.int32),
            pltpu.VMEM((_MSUB, _CH), jnp.int32),
            pltpu.VMEM((_MSUB, _CH), jnp.int32),
            pltpu.VMEM((_SUP, _H), jnp.float32),
            pltpu.VMEM((_SUP, _H), jnp.float32),
            pltpu.VMEM((_SUP, _H), jnp.float32),
            pltpu.VMEM((_SUP, _H), jnp.float32),
            pltpu.SemaphoreType.DMA,
            pltpu.SemaphoreType.DMA,
            pltpu.SemaphoreType.DMA,
            pltpu.SemaphoreType.DMA,
            pltpu.SemaphoreType.DMA,
            pltpu.SemaphoreType.DMA,
        ],
    )
    scatter = pl.kernel(
        _scatter_body,
        mesh=mesh,
        out_type=jax.ShapeDtypeStruct((_N, _H), jnp.float32),
        scratch_types=[
            pltpu.VMEM_SHARED((_NPC, _H), jnp.float32),
            pltpu.VMEM((_MSUB, _CH), jnp.int32),
            pltpu.VMEM((_SUP, _H), jnp.float32),
            pltpu.VMEM((_MSUB, _CH), jnp.int32),
            pltpu.VMEM((_SUP, _H), jnp.float32),
            pltpu.SemaphoreType.DMA,
            pltpu.SemaphoreType.DMA,
        ],
    )
    return gather, scatter


_f32 = jnp.float32


def _proj_call(xs, ws, wr):
    nb = _N // _BE
    return pl.pallas_call(
        _proj_body,
        grid=(nb,),
        in_specs=[
            pl.BlockSpec((_BE, _H), lambda i: (i, 0)),
            pl.BlockSpec((_H, _H), lambda i: (0, 0)),
            pl.BlockSpec((_H, _H), lambda i: (0, 0)),
        ],
        out_specs=[
            pl.BlockSpec((_BE, _H), lambda i: (i, 0)),
            pl.BlockSpec((_BE, _H), lambda i: (i, 0)),
        ],
        out_shape=[
            jax.ShapeDtypeStruct((_N, _H), _f32),
            jax.ShapeDtypeStruct((_N, _H), _f32),
        ],
    )(xs, ws, wr)


def _edge_call(es, g, w1, w2, w3, b2, b3, eb1):
    nb = _E // _BE
    bpg = _EPG // _BE
    return pl.pallas_call(
        _edge_body,
        grid=(nb,),
        in_specs=[
            pl.BlockSpec((_BE, _H), lambda i: (i, 0)),
            pl.BlockSpec((_BE, _H), lambda i: (i, 0)),
            pl.BlockSpec((_H, _H), lambda i: (0, 0)),
            pl.BlockSpec((_H, _H), lambda i: (0, 0)),
            pl.BlockSpec((_H, _H), lambda i: (0, 0)),
            pl.BlockSpec((1, _H), lambda i: (0, 0)),
            pl.BlockSpec((1, _H), lambda i: (0, 0)),
            pl.BlockSpec((1, 1, _H), lambda i: (i // bpg, 0, 0)),
        ],
        out_specs=pl.BlockSpec((_BE, _H), lambda i: (i, 0)),
        out_shape=jax.ShapeDtypeStruct((_E, _H), _f32),
    )(es, g, w1, w2, w3, b2, b3, eb1)


def _node_call(xs3, agg3, wa, wb, w2, w3, b2, b3, nb1):
    return pl.pallas_call(
        _node_body,
        grid=(_B,),
        in_specs=[
            pl.BlockSpec((1, _NPG, _H), lambda b: (b, 0, 0)),
            pl.BlockSpec((1, _NPG, _H), lambda b: (b, 0, 0)),
            pl.BlockSpec((_H, _H), lambda b: (0, 0)),
            pl.BlockSpec((_H, _H), lambda b: (0, 0)),
            pl.BlockSpec((_H, _H), lambda b: (0, 0)),
            pl.BlockSpec((_H, _H), lambda b: (0, 0)),
            pl.BlockSpec((1, _H), lambda b: (0, 0)),
            pl.BlockSpec((1, _H), lambda b: (0, 0)),
            pl.BlockSpec((1, 1, _H), lambda b: (b, 0, 0)),
        ],
        out_specs=pl.BlockSpec((1, _NPG, _H), lambda b: (b, 0, 0)),
        out_shape=jax.ShapeDtypeStruct((_B, _NPG, _H), _f32),
    )(xs3, agg3, wa, wb, w2, w3, b2, b3, nb1)


def _node_final_call(xs3, agg3, wa, wb, w2, w3, b2, b3, nb1, t2, inv):
    return pl.pallas_call(
        _node_final_body,
        grid=(_B,),
        in_specs=[
            pl.BlockSpec((1, _NPG, _H), lambda b: (b, 0, 0)),
            pl.BlockSpec((1, _NPG, _H), lambda b: (b, 0, 0)),
            pl.BlockSpec((_H, _H), lambda b: (0, 0)),
            pl.BlockSpec((_H, _H), lambda b: (0, 0)),
            pl.BlockSpec((_H, _H), lambda b: (0, 0)),
            pl.BlockSpec((_H, _H), lambda b: (0, 0)),
            pl.BlockSpec((1, _H), lambda b: (0, 0)),
            pl.BlockSpec((1, _H), lambda b: (0, 0)),
            pl.BlockSpec((1, 1, _H), lambda b: (b, 0, 0)),
            pl.BlockSpec((1, 1, _H), lambda b: (b, 0, 0)),
            pl.BlockSpec((1, 1, _H), lambda b: (b, 0, 0)),
        ],
        out_specs=pl.BlockSpec((1, _NPG, _H), lambda b: (b, 0, 0)),
        out_shape=jax.ShapeDtypeStruct((_B, _NPG, _H), _f32),
    )(xs3, agg3, wa, wb, w2, w3, b2, b3, nb1, t2, inv)


def _temb_call(t2d, gfp, twa, twb, tb, wsum, wn1, be, bn):
    return pl.pallas_call(
        _temb_body,
        out_shape=[
            jax.ShapeDtypeStruct((3, _B, _H), _f32),
            jax.ShapeDtypeStruct((3, _B, _H), _f32),
            jax.ShapeDtypeStruct((_B, _H), _f32),
            jax.ShapeDtypeStruct((_B, _H), _f32),
        ],
    )(t2d, gfp, twa, twb, tb, wsum, wn1, be, bn)


def kernel(x, edges, t, senders, receivers, params):
    lp = [params['layer%d' % l] for l in range(3)]
    w1e = [p['e_W'][0][0:_H] for p in lp]
    w1s = [p['e_W'][0][_H:2 * _H] for p in lp]
    w1r = [p['e_W'][0][2 * _H:3 * _H] for p in lp]
    wsum = jnp.stack([w1e[l] + w1s[l] + w1r[l] for l in range(3)])
    wn1a = [p['n_W'][0][0:_H] for p in lp]
    wn1b = [p['n_W'][0][_H:2 * _H] for p in lp]
    wn1 = jnp.stack(wn1a)
    gfp = jnp.stack([p['gfp_W'] for p in lp]).reshape(3, 1, _H // 2)
    twa = jnp.stack([p['t_W'][0:_H // 2] for p in lp])
    twb = jnp.stack([p['t_W'][_H // 2:] for p in lp])
    tb = jnp.stack([p['t_b'] for p in lp]).reshape(3, 1, _H)
    be = jnp.stack([p['e_b'][0] for p in lp]).reshape(3, 1, _H)
    bn = jnp.stack([p['n_b'][0] for p in lp]).reshape(3, 1, _H)
    t2d = t.reshape(_B, 1)

    eb1, nb1, temb2, inv = _temb_call(t2d, gfp, twa, twb, tb, wsum, wn1,
                                      be, bn)
    eb1_3 = [eb1[l].reshape(_B, 1, _H) for l in range(3)]
    nb1_3 = [nb1[l].reshape(_B, 1, _H) for l in range(3)]
    temb2_3 = temb2.reshape(_B, 1, _H)
    inv_3 = inv.reshape(_B, 1, _H)

    zro = jnp.zeros((_ZROWS, _H), _f32)
    snd3 = senders.astype(jnp.int32).reshape(_E // _SUP, _MSUB, _CH)
    rcv3 = receivers.astype(jnp.int32).reshape(_E // _SUP, _MSUB, _CH)
    rcv3l = rcv3 % _NPC

    gather_k, scatter_k = _sc_kernels()

    xs = x
    es = edges
    out3 = None
    for l in range(3):
        p = lp[l]
        p_s, p_r = _proj_call(xs, w1s[l], w1r[l])
        g = gather_k(p_s, p_r, snd3, rcv3)
        en = _edge_call(es, g, w1e[l], p['e_W'][1], p['e_W'][2],
                        p['e_b'][1].reshape(1, _H), p['e_b'][2].reshape(1, _H),
                        eb1_3[l])
        agg = scatter_k(en, rcv3l, zro)
        xs3 = xs.reshape(_B, _NPG, _H)
        agg3 = agg.reshape(_B, _NPG, _H)
        if l < 2:
            xs = _node_call(xs3, agg3, wn1a[l], wn1b[l], p['n_W'][1],
                            p['n_W'][2], p['n_b'][1].reshape(1, _H),
                            p['n_b'][2].reshape(1, _H),
                            nb1_3[l]).reshape(_N, _H)
        else:
            out3 = _node_final_call(xs3, agg3, wn1a[l], wn1b[l], p['n_W'][1],
                                    p['n_W'][2], p['n_b'][1].reshape(1, _H),
                                    p['n_b'][2].reshape(1, _H),
                                    nb1_3[l], temb2_3, inv_3)
        es = en
    return out3.reshape(_B, _NPG * _H)


# speedup vs baseline: 4.5904x; 1.1092x over previous
import functools

import numpy as np
import jax
import jax.numpy as jnp
from jax import lax
from jax.experimental import pallas as pl
from jax.experimental.pallas import tpu as pltpu
from jax.experimental.pallas import tpu_sc as plsc

_B = 8
_NPG = 1250
_EPG = 40000
_N = _B * _NPG
_E = _B * _EPG
_H = 128
_SIGMA = 25.0

_NC = 2
_NS = 16
_NW = _NC * _NS
_CH = 40
_MSUB = 5
_SUP = _CH * _MSUB
_EH = _E // 2
_NH = _N // 2
_EPW = _EH // _NW
_NSUP = _EPW // _SUP
_NPC = _NH // 2
_APAD = 2500
_ZROWS = 312
_ZTAIL = _APAD - 8 * _ZROWS

_BE = 2000
_BP = 1000


def _temb_body(t_ref, gfp_ref, twa_ref, twb_ref, tb_ref, wsum_ref, wn1_ref,
               be_ref, bn_ref, eb1_ref, nb1_ref, temb2_ref, inv_ref):
    t = t_ref[...]
    tembs = []
    for l in range(3):
        proj = t * gfp_ref[l] * (2.0 * np.pi)
        temb = (jnp.sin(proj) @ twa_ref[l]
                + jnp.cos(proj) @ twb_ref[l]
                + tb_ref[l])
        tembs.append(temb)
    for l in range(3):
        if l == 0:
            eb1_ref[0] = jnp.broadcast_to(be_ref[0], (_B, _H))
            nb1_ref[0] = jnp.broadcast_to(bn_ref[0], (_B, _H))
        else:
            eb1_ref[l] = tembs[l - 1] @ wsum_ref[l] + be_ref[l]
            nb1_ref[l] = tembs[l - 1] @ wn1_ref[l] + bn_ref[l]
    temb2_ref[...] = tembs[2]
    lsig = float(np.log(_SIGMA))
    var = (jnp.exp((2.0 * lsig) * t) - 1.0) / (2.0 * lsig)
    inv_ref[...] = lax.rsqrt(jnp.broadcast_to(var, (_B, _H)))


def _proj_body(xs_ref, ws_ref, wr_ref, outs_ref, outr_ref):
    x = xs_ref[...]
    outs_ref[...] = jnp.dot(x, ws_ref[...], preferred_element_type=jnp.float32)
    outr_ref[...] = jnp.dot(x, wr_ref[...], preferred_element_type=jnp.float32)


def _edge_body(es_ref, g_ref, w1_ref, w2_ref, w3_ref, b2_ref, b3_ref,
               b1_ref, out_ref):
    h = jnp.dot(es_ref[...], w1_ref[...], preferred_element_type=jnp.float32)
    h = jnp.maximum(h + g_ref[...] + b1_ref[0], 0.0)
    h = jnp.dot(h, w2_ref[...], preferred_element_type=jnp.float32)
    h = jnp.maximum(h + b2_ref[...], 0.0)
    out_ref[...] = (jnp.dot(h, w3_ref[...], preferred_element_type=jnp.float32)
                    + b3_ref[...])


def _node_body(xs_ref, agg_ref, wa_ref, wb_ref, w2_ref, w3_ref, b2_ref,
               b3_ref, b1_ref, out_ref):
    h = (jnp.dot(xs_ref[0], wa_ref[...], preferred_element_type=jnp.float32)
         + jnp.dot(agg_ref[0, 0], wb_ref[...],
                   preferred_element_type=jnp.float32)
         + b1_ref[0])
    h = jnp.maximum(h, 0.0)
    h = jnp.dot(h, w2_ref[...], preferred_element_type=jnp.float32)
    h = jnp.maximum(h + b2_ref[...], 0.0)
    out_ref[0] = (jnp.dot(h, w3_ref[...], preferred_element_type=jnp.float32)
                  + b3_ref[...])


def _node_final_body(xs_ref, agg_ref, wa_ref, wb_ref, w2_ref, w3_ref, b2_ref,
                     b3_ref, b1_ref, t2_ref, inv_ref, out_ref):
    h = (jnp.dot(xs_ref[0], wa_ref[...], preferred_element_type=jnp.float32)
         + jnp.dot(agg_ref[0, 0], wb_ref[...],
                   preferred_element_type=jnp.float32)
         + b1_ref[0])
    h = jnp.maximum(h, 0.0)
    h = jnp.dot(h, w2_ref[...], preferred_element_type=jnp.float32)
    h = jnp.maximum(h + b2_ref[...], 0.0)
    o = (jnp.dot(h, w3_ref[...], preferred_element_type=jnp.float32)
         + b3_ref[...] + t2_ref[0])
    out_ref[0] = o * inv_ref[0]


def _gather_body(ps, pr, snd3, rcv3, out,
                 idx_s0, idx_r0, idx_s1, idx_r1,
                 ba0, bb0, ba1, bb1,
                 sa0, sb0, sa1, sb1, sw0, sw1):
    c = lax.axis_index("c")
    s = lax.axis_index("s")
    q0 = (c * _NS + s) * _NSUP
    base0 = q0 * _SUP
    slots = ((idx_s0, idx_r0, ba0, bb0, sa0, sb0, sw0),
             (idx_s1, idx_r1, ba1, bb1, sa1, sb1, sw1))

    def load_fire(u, slot, wait_write):
        idx_s, idx_r, ba, bb, sa, sb, sw = slot
        pltpu.sync_copy(snd3.at[q0 + u], idx_s)
        pltpu.sync_copy(rcv3.at[q0 + u], idx_r)
        if wait_write:
            pltpu.make_async_copy(ba, out.at[pl.ds(0, _SUP)], sw).wait()
        for m in range(_MSUB):
            dst = pl.ds(m * _CH, _CH)
            pltpu.async_copy(ps.at[idx_s.at[m]], ba.at[dst], sa)
            pltpu.async_copy(pr.at[idx_r.at[m]], bb.at[dst], sb)

    def finish(u, slot):
        idx_s, idx_r, ba, bb, sa, sb, sw = slot
        base = base0 + u * _SUP
        for m in range(_MSUB):
            dst = pl.ds(m * _CH, _CH)
            pltpu.make_async_copy(ps.at[idx_s.at[m]], ba.at[dst], sa).wait()
            pltpu.make_async_copy(pr.at[idx_r.at[m]], bb.at[dst], sb).wait()

        def addrow(i, carry2):
            for k in range(_H // 16):
                sl = pl.ds(k * 16, 16)
                ba[i, sl] = ba[i, sl] + bb[i, sl]
            return carry2
        lax.fori_loop(0, _SUP, addrow, 0)
        pltpu.async_copy(ba, out.at[pl.ds(base, _SUP)], sw)

    load_fire(0, slots[0], False)
    load_fire(1, slots[1], False)

    def body(t, carry):
        u0 = 2 * t
        finish(u0, slots[0])

        @pl.when(u0 + 2 < _NSUP)
        def _refire0():
            load_fire(u0 + 2, slots[0], True)

        finish(u0 + 1, slots[1])

        @pl.when(u0 + 3 < _NSUP)
        def _refire1():
            load_fire(u0 + 3, slots[1], True)
        return carry
    lax.fori_loop(0, _NSUP // 2, body, 0)
    if _NSUP % 2:
        finish(_NSUP - 1, slots[0])

    pltpu.make_async_copy(ba0, out.at[pl.ds(0, _SUP)], sw0).wait()
    pltpu.make_async_copy(ba1, out.at[pl.ds(0, _SUP)], sw1).wait()


def _scatter_body(en, rcv3l, zro, agg, sh, idx0, dat0, idx1, dat1, ss0, ss1):
    c = lax.axis_index("c")
    s = lax.axis_index("s")

    @pl.when(s < 8)
    def _zero():
        pltpu.sync_copy(zro, sh.at[pl.ds(s * _ZROWS, _ZROWS)])

    @pl.when(s == 0)
    def _zero_tail():
        pltpu.sync_copy(zro.at[pl.ds(0, _ZTAIL)],
                        sh.at[pl.ds(8 * _ZROWS, _ZTAIL)])

    plsc.subcore_barrier()

    q0 = (c * _NS + s) * _NSUP
    base0 = q0 * _SUP
    slots = ((idx0, dat0, ss0), (idx1, dat1, ss1))

    def load_fire(u, slot):
        idx, dat, sem = slot
        base = base0 + u * _SUP
        pltpu.sync_copy(rcv3l.at[q0 + u], idx)
        pltpu.sync_copy(en.at[pl.ds(base, _SUP)], dat)
        for m in range(_MSUB):
            src = pl.ds(m * _CH, _CH)
            pltpu.async_copy(dat.at[src], sh.at[idx.at[m]], sem, add=True)

    def wait_sc(slot):
        idx, dat, sem = slot
        for m in range(_MSUB):
            src = pl.ds(m * _CH, _CH)
            pltpu.make_async_copy(dat.at[src], sh.at[idx.at[m]], sem).wait()

    load_fire(0, slots[0])

    def body(t, carry):
        u0 = 2 * t
        load_fire(u0 + 1, slots[1])
        wait_sc(slots[0])

        @pl.when(u0 + 2 < _NSUP)
        def _refire0():
            load_fire(u0 + 2, slots[0])
        wait_sc(slots[1])
        return carry
    lax.fori_loop(0, _NSUP // 2, body, 0)
    if _NSUP % 2:
        wait_sc(slots[0])

    plsc.subcore_barrier()

    @pl.when(s < 8)
    def _writeout():
        rows = pl.ds(s * _ZROWS, _ZROWS)
        pltpu.sync_copy(sh.at[rows], agg.at[c, rows])

    @pl.when(s == 0)
    def _write_tail():
        tail = pl.ds(8 * _ZROWS, _ZTAIL)
        pltpu.sync_copy(sh.at[tail], agg.at[c, tail])


@functools.cache
def _sc_kernels():
    mesh = plsc.VectorSubcoreMesh(core_axis_name="c", subcore_axis_name="s",
                                  num_cores=_NC, num_subcores=_NS)
    gather = pl.kernel(
        _gather_body,
        mesh=mesh,
        out_type=jax.ShapeDtypeStruct((_EH, _H), jnp.float32),
        scratch_types=[
            pltpu.VMEM((_MSUB, _CH), jnp.int32),
            pltpu.VMEM((_MSUB, _CH), jnp.int32),
            pltpu.VMEM((_MSUB, _CH), jnp.int32),
            pltpu.VMEM((_MSUB, _CH), jnp.int32),
            pltpu.VMEM((_SUP, _H), jnp.float32),
            pltpu.VMEM((_SUP, _H), jnp.float32),
            pltpu.VMEM((_SUP, _H), jnp.float32),
            pltpu.VMEM((_SUP, _H), jnp.float32),
            pltpu.SemaphoreType.DMA,
            pltpu.SemaphoreType.DMA,
            pltpu.SemaphoreType.DMA,
            pltpu.SemaphoreType.DMA,
            pltpu.SemaphoreType.DMA,
            pltpu.SemaphoreType.DMA,
        ],
    )
    scatter = pl.kernel(
        _scatter_body,
        mesh=mesh,
        out_type=jax.ShapeDtypeStruct((2, _APAD, _H), jnp.float32),
        scratch_types=[
            pltpu.VMEM_SHARED((_APAD, _H), jnp.float32),
            pltpu.VMEM((_MSUB, _CH), jnp.int32),
            pltpu.VMEM((_SUP, _H), jnp.float32),
            pltpu.VMEM((_MSUB, _CH), jnp.int32),
            pltpu.VMEM((_SUP, _H), jnp.float32),
            pltpu.SemaphoreType.DMA,
            pltpu.SemaphoreType.DMA,
        ],
    )
    return gather, scatter


_f32 = jnp.float32


def _proj_call(xs, ws, wr):
    nb = _NH // _BP
    return pl.pallas_call(
        _proj_body,
        grid=(nb,),
        in_specs=[
            pl.BlockSpec((_BP, _H), lambda i: (i, 0)),
            pl.BlockSpec((_H, _H), lambda i: (0, 0)),
            pl.BlockSpec((_H, _H), lambda i: (0, 0)),
        ],
        out_specs=[
            pl.BlockSpec((_BP, _H), lambda i: (i, 0)),
            pl.BlockSpec((_BP, _H), lambda i: (i, 0)),
        ],
        out_shape=[
            jax.ShapeDtypeStruct((_NH, _H), _f32),
            jax.ShapeDtypeStruct((_NH, _H), _f32),
        ],
    )(xs, ws, wr)


def _edge_call(es, g, w1, w2, w3, b2, b3, eb1):
    nb = _EH // _BE
    bpg = _EPG // _BE
    return pl.pallas_call(
        _edge_body,
        grid=(nb,),
        in_specs=[
            pl.BlockSpec((_BE, _H), lambda i: (i, 0)),
            pl.BlockSpec((_BE, _H), lambda i: (i, 0)),
            pl.BlockSpec((_H, _H), lambda i: (0, 0)),
            pl.BlockSpec((_H, _H), lambda i: (0, 0)),
            pl.BlockSpec((_H, _H), lambda i: (0, 0)),
            pl.BlockSpec((1, _H), lambda i: (0, 0)),
            pl.BlockSpec((1, _H), lambda i: (0, 0)),
            pl.BlockSpec((1, 1, _H), lambda i: (i // bpg, 0, 0)),
        ],
        out_specs=pl.BlockSpec((_BE, _H), lambda i: (i, 0)),
        out_shape=jax.ShapeDtypeStruct((_EH, _H), _f32),
    )(es, g, w1, w2, w3, b2, b3, eb1)


def _node_call(xs3, agg3, wa, wb, w2, w3, b2, b3, nb1):
    return pl.pallas_call(
        _node_body,
        grid=(_B // 2,),
        in_specs=[
            pl.BlockSpec((1, _NPG, _H), lambda b: (b, 0, 0)),
            pl.BlockSpec((1, 1, _NPG, _H), lambda b: (b // 2, b % 2, 0, 0)),
            pl.BlockSpec((_H, _H), lambda b: (0, 0)),
            pl.BlockSpec((_H, _H), lambda b: (0, 0)),
            pl.BlockSpec((_H, _H), lambda b: (0, 0)),
            pl.BlockSpec((_H, _H), lambda b: (0, 0)),
            pl.BlockSpec((1, _H), lambda b: (0, 0)),
            pl.BlockSpec((1, _H), lambda b: (0, 0)),
            pl.BlockSpec((1, 1, _H), lambda b: (b, 0, 0)),
        ],
        out_specs=pl.BlockSpec((1, _NPG, _H), lambda b: (b, 0, 0)),
        out_shape=jax.ShapeDtypeStruct((_B // 2, _NPG, _H), _f32),
    )(xs3, agg3, wa, wb, w2, w3, b2, b3, nb1)


def _node_final_call(xs3, agg3, wa, wb, w2, w3, b2, b3, nb1, t2, inv):
    return pl.pallas_call(
        _node_final_body,
        grid=(_B // 2,),
        in_specs=[
            pl.BlockSpec((1, _NPG, _H), lambda b: (b, 0, 0)),
            pl.BlockSpec((1, 1, _NPG, _H), lambda b: (b // 2, b % 2, 0, 0)),
            pl.BlockSpec((_H, _H), lambda b: (0, 0)),
            pl.BlockSpec((_H, _H), lambda b: (0, 0)),
            pl.BlockSpec((_H, _H), lambda b: (0, 0)),
            pl.BlockSpec((_H, _H), lambda b: (0, 0)),
            pl.BlockSpec((1, _H), lambda b: (0, 0)),
            pl.BlockSpec((1, _H), lambda b: (0, 0)),
            pl.BlockSpec((1, 1, _H), lambda b: (b, 0, 0)),
            pl.BlockSpec((1, 1, _H), lambda b: (b, 0, 0)),
            pl.BlockSpec((1, 1, _H), lambda b: (b, 0, 0)),
        ],
        out_specs=pl.BlockSpec((1, _NPG, _H), lambda b: (b, 0, 0)),
        out_shape=jax.ShapeDtypeStruct((_B // 2, _NPG, _H), _f32),
    )(xs3, agg3, wa, wb, w2, w3, b2, b3, nb1, t2, inv)


def _temb_call(t2d, gfp, twa, twb, tb, wsum, wn1, be, bn):
    return pl.pallas_call(
        _temb_body,
        out_shape=[
            jax.ShapeDtypeStruct((3, _B, _H), _f32),
            jax.ShapeDtypeStruct((3, _B, _H), _f32),
            jax.ShapeDtypeStruct((_B, _H), _f32),
            jax.ShapeDtypeStruct((_B, _H), _f32),
        ],
    )(t2d, gfp, twa, twb, tb, wsum, wn1, be, bn)


def kernel(x, edges, t, senders, receivers, params):
    lp = [params['layer%d' % l] for l in range(3)]
    w1e = [p['e_W'][0][0:_H] for p in lp]
    w1s = [p['e_W'][0][_H:2 * _H] for p in lp]
    w1r = [p['e_W'][0][2 * _H:3 * _H] for p in lp]
    wsum = jnp.stack([w1e[l] + w1s[l] + w1r[l] for l in range(3)])
    wn1a = [p['n_W'][0][0:_H] for p in lp]
    wn1b = [p['n_W'][0][_H:2 * _H] for p in lp]
    wn1 = jnp.stack(wn1a)
    gfp = jnp.stack([p['gfp_W'] for p in lp]).reshape(3, 1, _H // 2)
    twa = jnp.stack([p['t_W'][0:_H // 2] for p in lp])
    twb = jnp.stack([p['t_W'][_H // 2:] for p in lp])
    tb = jnp.stack([p['t_b'] for p in lp]).reshape(3, 1, _H)
    be = jnp.stack([p['e_b'][0] for p in lp]).reshape(3, 1, _H)
    bn = jnp.stack([p['n_b'][0] for p in lp]).reshape(3, 1, _H)
    t2d = t.reshape(_B, 1)

    eb1, nb1, temb2, inv = _temb_call(t2d, gfp, twa, twb, tb, wsum, wn1,
                                      be, bn)
    eb1_3 = [eb1[l].reshape(_B, 1, _H) for l in range(3)]
    nb1_3 = [nb1[l].reshape(_B, 1, _H) for l in range(3)]
    temb2_3 = temb2.reshape(_B, 1, _H)
    inv_3 = inv.reshape(_B, 1, _H)

    zro = jnp.zeros((_ZROWS, _H), _f32)
    snd_l = (senders.astype(jnp.int32) % _NH).reshape(-1, _MSUB, _CH)
    rcv_g = receivers.astype(jnp.int32)
    rcv_l = (rcv_g % _NH).reshape(-1, _MSUB, _CH)
    rcv_c = (rcv_g % _NPC).reshape(-1, _MSUB, _CH)
    nsup_h = _EH // _SUP
    snd3 = [snd_l[h * nsup_h:(h + 1) * nsup_h] for h in range(2)]
    rcv3 = [rcv_l[h * nsup_h:(h + 1) * nsup_h] for h in range(2)]
    rcv3c = [rcv_c[h * nsup_h:(h + 1) * nsup_h] for h in range(2)]

    gather_k, scatter_k = _sc_kernels()

    xs = [x[:_NH], x[_NH:]]
    es = [edges[:_EH], edges[_EH:]]
    out3 = [None, None]
    for l in range(3):
        p = lp[l]
        eb2 = p['e_b'][1].reshape(1, _H)
        eb3 = p['e_b'][2].reshape(1, _H)
        nb2 = p['n_b'][1].reshape(1, _H)
        nb3 = p['n_b'][2].reshape(1, _H)
        for h in range(2):
            p_s, p_r = _proj_call(xs[h], w1s[l], w1r[l])
            g = gather_k(p_s, p_r, snd3[h], rcv3[h])
            en = _edge_call(es[h], g, w1e[l], p['e_W'][1], p['e_W'][2],
                            eb2, eb3, eb1_3[l][4 * h:4 * h + 4])
            agg = scatter_k(en, rcv3c[h], zro).reshape(2, 2, _NPG, _H)
            xs3 = xs[h].reshape(_B // 2, _NPG, _H)
            if l < 2:
                xs[h] = _node_call(xs3, agg, wn1a[l], wn1b[l], p['n_W'][1],
                                   p['n_W'][2], nb2, nb3,
                                   nb1_3[l][4 * h:4 * h + 4]).reshape(_NH, _H)
            else:
                out3[h] = _node_final_call(
                    xs3, agg, wn1a[l], wn1b[l], p['n_W'][1], p['n_W'][2],
                    nb2, nb3, nb1_3[l][4 * h:4 * h + 4],
                    temb2_3[4 * h:4 * h + 4], inv_3[4 * h:4 * h + 4])
            es[h] = en
    return jnp.concatenate([out3[0].reshape(_B // 2, _NPG * _H),
                            out3[1].reshape(_B // 2, _NPG * _H)], axis=0)


# speedup vs baseline: 4.6784x; 1.0192x over previous
import functools

import numpy as np
import jax
import jax.numpy as jnp
from jax import lax
from jax.experimental import pallas as pl
from jax.experimental.pallas import tpu as pltpu
from jax.experimental.pallas import tpu_sc as plsc

_B = 8
_NPG = 1250
_EPG = 40000
_N = _B * _NPG
_E = _B * _EPG
_H = 128
_SIGMA = 25.0

_NC = 2
_NS = 16
_NW = _NC * _NS
_CH = 40
_MSUB = 5
_SUP = _CH * _MSUB
_EH = _E // 2
_NH = _N // 2
_EPW = _EH // _NW
_NSUP = _EPW // _SUP
_NPC = _NH // 2
_APAD = 2500
_ZROWS = 312
_ZTAIL = _APAD - 8 * _ZROWS

_BE = 2000
_BP = 1000


def _temb_body(t_ref, gfp_ref, twa_ref, twb_ref, tb_ref, wsum_ref, wn1_ref,
               be_ref, bn_ref, eb1_ref, nb1_ref, temb2_ref, inv_ref):
    t = t_ref[...]
    tembs = []
    for l in range(3):
        proj = t * gfp_ref[l] * (2.0 * np.pi)
        temb = (jnp.sin(proj) @ twa_ref[l]
                + jnp.cos(proj) @ twb_ref[l]
                + tb_ref[l])
        tembs.append(temb)
    for l in range(3):
        if l == 0:
            eb1_ref[0] = jnp.broadcast_to(be_ref[0], (_B, _H))
            nb1_ref[0] = jnp.broadcast_to(bn_ref[0], (_B, _H))
        else:
            eb1_ref[l] = tembs[l - 1] @ wsum_ref[l] + be_ref[l]
            nb1_ref[l] = tembs[l - 1] @ wn1_ref[l] + bn_ref[l]
    temb2_ref[...] = tembs[2]
    lsig = float(np.log(_SIGMA))
    var = (jnp.exp((2.0 * lsig) * t) - 1.0) / (2.0 * lsig)
    inv_ref[...] = lax.rsqrt(jnp.broadcast_to(var, (_B, _H)))


def _proj_body(xs_ref, ws_ref, wr_ref, outs_ref, outr_ref):
    x = xs_ref[...]
    outs_ref[...] = jnp.dot(x, ws_ref[...], preferred_element_type=jnp.float32)
    outr_ref[...] = jnp.dot(x, wr_ref[...], preferred_element_type=jnp.float32)


def _edge_body(es_ref, g_ref, w1_ref, w2_ref, w3_ref, b2_ref, b3_ref,
               b1_ref, out_ref):
    h = jnp.dot(es_ref[...], w1_ref[...], preferred_element_type=jnp.float32)
    h = jnp.maximum(h + g_ref[...] + b1_ref[0], 0.0)
    h = jnp.dot(h, w2_ref[...], preferred_element_type=jnp.float32)
    h = jnp.maximum(h + b2_ref[...], 0.0)
    out_ref[...] = (jnp.dot(h, w3_ref[...], preferred_element_type=jnp.float32)
                    + b3_ref[...])


def _node_body(xs_ref, agg_ref, wa_ref, wb_ref, w2_ref, w3_ref, b2_ref,
               b3_ref, b1_ref, out_ref):
    h = (jnp.dot(xs_ref[0], wa_ref[...], preferred_element_type=jnp.float32)
         + jnp.dot(agg_ref[0, 0], wb_ref[...],
                   preferred_element_type=jnp.float32)
         + b1_ref[0])
    h = jnp.maximum(h, 0.0)
    h = jnp.dot(h, w2_ref[...], preferred_element_type=jnp.float32)
    h = jnp.maximum(h + b2_ref[...], 0.0)
    out_ref[0] = (jnp.dot(h, w3_ref[...], preferred_element_type=jnp.float32)
                  + b3_ref[...])


def _node_final_body(xs_ref, agg_ref, wa_ref, wb_ref, w2_ref, w3_ref, b2_ref,
                     b3_ref, b1_ref, t2_ref, inv_ref, out_ref):
    h = (jnp.dot(xs_ref[0], wa_ref[...], preferred_element_type=jnp.float32)
         + jnp.dot(agg_ref[0, 0], wb_ref[...],
                   preferred_element_type=jnp.float32)
         + b1_ref[0])
    h = jnp.maximum(h, 0.0)
    h = jnp.dot(h, w2_ref[...], preferred_element_type=jnp.float32)
    h = jnp.maximum(h + b2_ref[...], 0.0)
    o = (jnp.dot(h, w3_ref[...], preferred_element_type=jnp.float32)
         + b3_ref[...] + t2_ref[0])
    out_ref[0] = o * inv_ref[0]


def _gather_body(ps, pr, snd3, rcv3, out,
                 idx_s, idx_r0, idx_r1,
                 ba0, bb0, ba1, bb1,
                 sa0, sb0, sa1, sb1, sw0, sw1):
    c = lax.axis_index("c")
    s = lax.axis_index("s")
    q0 = (c * _NS + s) * _NSUP
    base0 = q0 * _SUP
    slots = ((idx_r0, ba0, bb0, sa0, sb0, sw0),
             (idx_r1, ba1, bb1, sa1, sb1, sw1))

    pltpu.sync_copy(snd3.at[pl.ds(q0, _NSUP)], idx_s)

    def load_fire(u, slot, wait_write):
        idx_r, ba, bb, sa, sb, sw = slot
        pltpu.sync_copy(rcv3.at[q0 + u], idx_r)
        if wait_write:
            pltpu.make_async_copy(ba, out.at[pl.ds(0, _SUP)], sw).wait()
        for m in range(_MSUB):
            dst = pl.ds(m * _CH, _CH)
            pltpu.async_copy(ps.at[idx_s.at[u, m]], ba.at[dst], sa)
            pltpu.async_copy(pr.at[idx_r.at[m]], bb.at[dst], sb)

    def finish(u, slot):
        idx_r, ba, bb, sa, sb, sw = slot
        base = base0 + u * _SUP
        for m in range(_MSUB):
            dst = pl.ds(m * _CH, _CH)
            pltpu.make_async_copy(ps.at[idx_s.at[u, m]], ba.at[dst],
                                  sa).wait()
            pltpu.make_async_copy(pr.at[idx_r.at[m]], bb.at[dst],
                                  sb).wait()

        def addrow(i, carry2):
            for k in range(_H // 16):
                sl = pl.ds(k * 16, 16)
                ba[i, sl] = ba[i, sl] + bb[i, sl]
            return carry2
        lax.fori_loop(0, _SUP, addrow, 0)
        pltpu.async_copy(ba, out.at[pl.ds(base, _SUP)], sw)

    load_fire(0, slots[0], False)
    load_fire(1, slots[1], False)

    def body(t, carry):
        u0 = 2 * t
        finish(u0, slots[0])

        @pl.when(u0 + 2 < _NSUP)
        def _refire0():
            load_fire(u0 + 2, slots[0], True)

        finish(u0 + 1, slots[1])

        @pl.when(u0 + 3 < _NSUP)
        def _refire1():
            load_fire(u0 + 3, slots[1], True)
        return carry
    lax.fori_loop(0, _NSUP // 2, body, 0)
    if _NSUP % 2:
        finish(_NSUP - 1, slots[0])

    pltpu.make_async_copy(ba0, out.at[pl.ds(0, _SUP)], sw0).wait()
    pltpu.make_async_copy(ba1, out.at[pl.ds(0, _SUP)], sw1).wait()


def _scatter_body(en, rcv3l, zro, agg, sh, idx_a, dat0, dat1, ss0, ss1):
    c = lax.axis_index("c")
    s = lax.axis_index("s")

    @pl.when(s < 8)
    def _zero():
        pltpu.sync_copy(zro, sh.at[pl.ds(s * _ZROWS, _ZROWS)])

    @pl.when(s == 0)
    def _zero_tail():
        pltpu.sync_copy(zro.at[pl.ds(0, _ZTAIL)],
                        sh.at[pl.ds(8 * _ZROWS, _ZTAIL)])

    plsc.subcore_barrier()

    q0 = (c * _NS + s) * _NSUP
    base0 = q0 * _SUP
    slots = ((dat0, ss0), (dat1, ss1))

    pltpu.sync_copy(rcv3l.at[pl.ds(q0, _NSUP)], idx_a)

    def load_fire(u, slot):
        dat, sem = slot
        base = base0 + u * _SUP
        pltpu.sync_copy(en.at[pl.ds(base, _SUP)], dat)
        for m in range(_MSUB):
            src = pl.ds(m * _CH, _CH)
            pltpu.async_copy(dat.at[src], sh.at[idx_a.at[u, m]], sem,
                             add=True)

    def wait_sc(slot):
        dat, sem = slot
        for m in range(_MSUB):
            src = pl.ds(m * _CH, _CH)
            pltpu.make_async_copy(dat.at[src], sh.at[idx_a.at[0, m]],
                                  sem).wait()

    load_fire(0, slots[0])

    def body(t, carry):
        u0 = 2 * t
        load_fire(u0 + 1, slots[1])
        wait_sc(slots[0])

        @pl.when(u0 + 2 < _NSUP)
        def _refire0():
            load_fire(u0 + 2, slots[0])
        wait_sc(slots[1])
        return carry
    lax.fori_loop(0, _NSUP // 2, body, 0)
    if _NSUP % 2:
        wait_sc(slots[0])

    plsc.subcore_barrier()

    @pl.when(s < 8)
    def _writeout():
        rows = pl.ds(s * _ZROWS, _ZROWS)
        pltpu.sync_copy(sh.at[rows], agg.at[c, rows])

    @pl.when(s == 0)
    def _write_tail():
        tail = pl.ds(8 * _ZROWS, _ZTAIL)
        pltpu.sync_copy(sh.at[tail], agg.at[c, tail])


@functools.cache
def _sc_kernels():
    mesh = plsc.VectorSubcoreMesh(core_axis_name="c", subcore_axis_name="s",
                                  num_cores=_NC, num_subcores=_NS)
    gather = pl.kernel(
        _gather_body,
        mesh=mesh,
        out_type=jax.ShapeDtypeStruct((_EH, _H), jnp.float32),
        scratch_types=[
            pltpu.VMEM((_NSUP, _MSUB, _CH), jnp.int32),
            pltpu.VMEM((_MSUB, _CH), jnp.int32),
            pltpu.VMEM((_MSUB, _CH), jnp.int32),
            pltpu.VMEM((_SUP, _H), jnp.float32),
            pltpu.VMEM((_SUP, _H), jnp.float32),
            pltpu.VMEM((_SUP, _H), jnp.float32),
            pltpu.VMEM((_SUP, _H), jnp.float32),
            pltpu.SemaphoreType.DMA,
            pltpu.SemaphoreType.DMA,
            pltpu.SemaphoreType.DMA,
            pltpu.SemaphoreType.DMA,
            pltpu.SemaphoreType.DMA,
            pltpu.SemaphoreType.DMA,
        ],
    )
    scatter = pl.kernel(
        _scatter_body,
        mesh=mesh,
        out_type=jax.ShapeDtypeStruct((2, _APAD, _H), jnp.float32),
        scratch_types=[
            pltpu.VMEM_SHARED((_APAD, _H), jnp.float32),
            pltpu.VMEM((_NSUP, _MSUB, _CH), jnp.int32),
            pltpu.VMEM((_SUP, _H), jnp.float32),
            pltpu.VMEM((_SUP, _H), jnp.float32),
            pltpu.SemaphoreType.DMA,
            pltpu.SemaphoreType.DMA,
        ],
    )
    return gather, scatter


_f32 = jnp.float32


def _proj_call(xs, ws, wr):
    nb = _NH // _BP
    return pl.pallas_call(
        _proj_body,
        grid=(nb,),
        in_specs=[
            pl.BlockSpec((_BP, _H), lambda i: (i, 0)),
            pl.BlockSpec((_H, _H), lambda i: (0, 0)),
            pl.BlockSpec((_H, _H), lambda i: (0, 0)),
        ],
        out_specs=[
            pl.BlockSpec((_BP, _H), lambda i: (i, 0)),
            pl.BlockSpec((_BP, _H), lambda i: (i, 0)),
        ],
        out_shape=[
            jax.ShapeDtypeStruct((_NH, _H), _f32),
            jax.ShapeDtypeStruct((_NH, _H), _f32),
        ],
    )(xs, ws, wr)


def _edge_call(es, g, w1, w2, w3, b2, b3, eb1):
    nb = _EH // _BE
    bpg = _EPG // _BE
    return pl.pallas_call(
        _edge_body,
        grid=(nb,),
        in_specs=[
            pl.BlockSpec((_BE, _H), lambda i: (i, 0)),
            pl.BlockSpec((_BE, _H), lambda i: (i, 0)),
            pl.BlockSpec((_H, _H), lambda i: (0, 0)),
            pl.BlockSpec((_H, _H), lambda i: (0, 0)),
            pl.BlockSpec((_H, _H), lambda i: (0, 0)),
            pl.BlockSpec((1, _H), lambda i: (0, 0)),
            pl.BlockSpec((1, _H), lambda i: (0, 0)),
            pl.BlockSpec((1, 1, _H), lambda i: (i // bpg, 0, 0)),
        ],
        out_specs=pl.BlockSpec((_BE, _H), lambda i: (i, 0)),
        out_shape=jax.ShapeDtypeStruct((_EH, _H), _f32),
    )(es, g, w1, w2, w3, b2, b3, eb1)


def _node_call(xs3, agg3, wa, wb, w2, w3, b2, b3, nb1):
    return pl.pallas_call(
        _node_body,
        grid=(_B // 2,),
        in_specs=[
            pl.BlockSpec((1, _NPG, _H), lambda b: (b, 0, 0)),
            pl.BlockSpec((1, 1, _NPG, _H), lambda b: (b // 2, b % 2, 0, 0)),
            pl.BlockSpec((_H, _H), lambda b: (0, 0)),
            pl.BlockSpec((_H, _H), lambda b: (0, 0)),
            pl.BlockSpec((_H, _H), lambda b: (0, 0)),
            pl.BlockSpec((_H, _H), lambda b: (0, 0)),
            pl.BlockSpec((1, _H), lambda b: (0, 0)),
            pl.BlockSpec((1, _H), lambda b: (0, 0)),
            pl.BlockSpec((1, 1, _H), lambda b: (b, 0, 0)),
        ],
        out_specs=pl.BlockSpec((1, _NPG, _H), lambda b: (b, 0, 0)),
        out_shape=jax.ShapeDtypeStruct((_B // 2, _NPG, _H), _f32),
    )(xs3, agg3, wa, wb, w2, w3, b2, b3, nb1)


def _node_final_call(xs3, agg3, wa, wb, w2, w3, b2, b3, nb1, t2, inv):
    return pl.pallas_call(
        _node_final_body,
        grid=(_B // 2,),
        in_specs=[
            pl.BlockSpec((1, _NPG, _H), lambda b: (b, 0, 0)),
            pl.BlockSpec((1, 1, _NPG, _H), lambda b: (b // 2, b % 2, 0, 0)),
            pl.BlockSpec((_H, _H), lambda b: (0, 0)),
            pl.BlockSpec((_H, _H), lambda b: (0, 0)),
            pl.BlockSpec((_H, _H), lambda b: (0, 0)),
            pl.BlockSpec((_H, _H), lambda b: (0, 0)),
            pl.BlockSpec((1, _H), lambda b: (0, 0)),
            pl.BlockSpec((1, _H), lambda b: (0, 0)),
            pl.BlockSpec((1, 1, _H), lambda b: (b, 0, 0)),
            pl.BlockSpec((1, 1, _H), lambda b: (b, 0, 0)),
            pl.BlockSpec((1, 1, _H), lambda b: (b, 0, 0)),
        ],
        out_specs=pl.BlockSpec((1, _NPG, _H), lambda b: (b, 0, 0)),
        out_shape=jax.ShapeDtypeStruct((_B // 2, _NPG, _H), _f32),
    )(xs3, agg3, wa, wb, w2, w3, b2, b3, nb1, t2, inv)


def _temb_call(t2d, gfp, twa, twb, tb, wsum, wn1, be, bn):
    return pl.pallas_call(
        _temb_body,
        out_shape=[
            jax.ShapeDtypeStruct((3, _B, _H), _f32),
            jax.ShapeDtypeStruct((3, _B, _H), _f32),
            jax.ShapeDtypeStruct((_B, _H), _f32),
            jax.ShapeDtypeStruct((_B, _H), _f32),
        ],
    )(t2d, gfp, twa, twb, tb, wsum, wn1, be, bn)


def kernel(x, edges, t, senders, receivers, params):
    lp = [params['layer%d' % l] for l in range(3)]
    w1e = [p['e_W'][0][0:_H] for p in lp]
    w1s = [p['e_W'][0][_H:2 * _H] for p in lp]
    w1r = [p['e_W'][0][2 * _H:3 * _H] for p in lp]
    wsum = jnp.stack([w1e[l] + w1s[l] + w1r[l] for l in range(3)])
    wn1a = [p['n_W'][0][0:_H] for p in lp]
    wn1b = [p['n_W'][0][_H:2 * _H] for p in lp]
    wn1 = jnp.stack(wn1a)
    gfp = jnp.stack([p['gfp_W'] for p in lp]).reshape(3, 1, _H // 2)
    twa = jnp.stack([p['t_W'][0:_H // 2] for p in lp])
    twb = jnp.stack([p['t_W'][_H // 2:] for p in lp])
    tb = jnp.stack([p['t_b'] for p in lp]).reshape(3, 1, _H)
    be = jnp.stack([p['e_b'][0] for p in lp]).reshape(3, 1, _H)
    bn = jnp.stack([p['n_b'][0] for p in lp]).reshape(3, 1, _H)
    t2d = t.reshape(_B, 1)

    eb1, nb1, temb2, inv = _temb_call(t2d, gfp, twa, twb, tb, wsum, wn1,
                                      be, bn)
    eb1_3 = [eb1[l].reshape(_B, 1, _H) for l in range(3)]
    nb1_3 = [nb1[l].reshape(_B, 1, _H) for l in range(3)]
    temb2_3 = temb2.reshape(_B, 1, _H)
    inv_3 = inv.reshape(_B, 1, _H)

    zro = jnp.zeros((_ZROWS, _H), _f32)
    snd_l = (senders.astype(jnp.int32) % _NH).reshape(-1, _MSUB, _CH)
    rcv_g = receivers.astype(jnp.int32)
    rcv_l = (rcv_g % _NH).reshape(-1, _MSUB, _CH)
    rcv_c = (rcv_g % _NPC).reshape(-1, _MSUB, _CH)
    nsup_h = _EH // _SUP
    snd3 = [snd_l[h * nsup_h:(h + 1) * nsup_h] for h in range(2)]
    rcv3 = [rcv_l[h * nsup_h:(h + 1) * nsup_h] for h in range(2)]
    rcv3c = [rcv_c[h * nsup_h:(h + 1) * nsup_h] for h in range(2)]

    gather_k, scatter_k = _sc_kernels()

    xs = [x[:_NH], x[_NH:]]
    es = [edges[:_EH], edges[_EH:]]
    out3 = [None, None]
    for l in range(3):
        p = lp[l]
        eb2 = p['e_b'][1].reshape(1, _H)
        eb3 = p['e_b'][2].reshape(1, _H)
        nb2 = p['n_b'][1].reshape(1, _H)
        nb3 = p['n_b'][2].reshape(1, _H)
        for h in range(2):
            p_s, p_r = _proj_call(xs[h], w1s[l], w1r[l])
            g = gather_k(p_s, p_r, snd3[h], rcv3[h])
            en = _edge_call(es[h], g, w1e[l], p['e_W'][1], p['e_W'][2],
                            eb2, eb3, eb1_3[l][4 * h:4 * h + 4])
            agg = scatter_k(en, rcv3c[h], zro).reshape(2, 2, _NPG, _H)
            xs3 = xs[h].reshape(_B // 2, _NPG, _H)
            if l < 2:
                xs[h] = _node_call(xs3, agg, wn1a[l], wn1b[l], p['n_W'][1],
                                   p['n_W'][2], nb2, nb3,
                                   nb1_3[l][4 * h:4 * h + 4]).reshape(_NH, _H)
            else:
                out3[h] = _node_final_call(
                    xs3, agg, wn1a[l], wn1b[l], p['n_W'][1], p['n_W'][2],
                    nb2, nb3, nb1_3[l][4 * h:4 * h + 4],
                    temb2_3[4 * h:4 * h + 4], inv_3[4 * h:4 * h + 4])
            es[h] = en
    return jnp.concatenate([out3[0].reshape(_B // 2, _NPG * _H),
                            out3[1].reshape(_B // 2, _NPG * _H)], axis=0)


# speedup vs baseline: 4.7019x; 1.0050x over previous
import functools

import numpy as np
import jax
import jax.numpy as jnp
from jax import lax
from jax.experimental import pallas as pl
from jax.experimental.pallas import tpu as pltpu
from jax.experimental.pallas import tpu_sc as plsc

_B = 8
_NPG = 1250
_EPG = 40000
_N = _B * _NPG
_E = _B * _EPG
_H = 128
_SIGMA = 25.0

_NC = 2
_NS = 16
_NW = _NC * _NS
_CH = 40
_MSUB = 5
_SUP = _CH * _MSUB
_EH = _E // 2
_NH = _N // 2
_EPW = _EH // _NW
_NSUP = _EPW // _SUP
_NPC = _NH // 2
_APAD = 2500
_ZROWS = 312
_ZTAIL = _APAD - 8 * _ZROWS

_BE = 2000
_BP = 1000


def _temb_body(t_ref, gfp_ref, twa_ref, twb_ref, tb_ref, wsum_ref, wn1_ref,
               be_ref, bn_ref, eb1_ref, nb1_ref, temb2_ref, inv_ref):
    t = t_ref[...]
    tembs = []
    for l in range(3):
        proj = t * gfp_ref[l] * (2.0 * np.pi)
        temb = (jnp.sin(proj) @ twa_ref[l]
                + jnp.cos(proj) @ twb_ref[l]
                + tb_ref[l])
        tembs.append(temb)
    for l in range(3):
        if l == 0:
            eb1_ref[0] = jnp.broadcast_to(be_ref[0], (_B, _H))
            nb1_ref[0] = jnp.broadcast_to(bn_ref[0], (_B, _H))
        else:
            eb1_ref[l] = tembs[l - 1] @ wsum_ref[l] + be_ref[l]
            nb1_ref[l] = tembs[l - 1] @ wn1_ref[l] + bn_ref[l]
    temb2_ref[...] = tembs[2]
    lsig = float(np.log(_SIGMA))
    var = (jnp.exp((2.0 * lsig) * t) - 1.0) / (2.0 * lsig)
    inv_ref[...] = lax.rsqrt(jnp.broadcast_to(var, (_B, _H)))


def _proj_body(xs_ref, ws_ref, wr_ref, outs_ref, outr_ref):
    x = xs_ref[...]
    outs_ref[...] = jnp.dot(x, ws_ref[...], preferred_element_type=jnp.float32)
    outr_ref[...] = jnp.dot(x, wr_ref[...], preferred_element_type=jnp.float32)


def _edge_body(es_ref, g_ref, w1_ref, w2_ref, w3_ref, b2_ref, b3_ref,
               b1_ref, out_ref):
    h = jnp.dot(es_ref[...], w1_ref[...], preferred_element_type=jnp.float32)
    h = jnp.maximum(h + g_ref[...] + b1_ref[0], 0.0)
    h = jnp.dot(h, w2_ref[...], preferred_element_type=jnp.float32)
    h = jnp.maximum(h + b2_ref[...], 0.0)
    out_ref[...] = (jnp.dot(h, w3_ref[...], preferred_element_type=jnp.float32)
                    + b3_ref[...])


def _node_body(xs_ref, agg_ref, wa_ref, wb_ref, w2_ref, w3_ref, b2_ref,
               b3_ref, b1_ref, ws_ref, wr_ref, out_ref, ps_ref, pr_ref):
    h = (jnp.dot(xs_ref[0], wa_ref[...], preferred_element_type=jnp.float32)
         + jnp.dot(agg_ref[0, 0], wb_ref[...],
                   preferred_element_type=jnp.float32)
         + b1_ref[0])
    h = jnp.maximum(h, 0.0)
    h = jnp.dot(h, w2_ref[...], preferred_element_type=jnp.float32)
    h = jnp.maximum(h + b2_ref[...], 0.0)
    o = (jnp.dot(h, w3_ref[...], preferred_element_type=jnp.float32)
         + b3_ref[...])
    out_ref[0] = o
    ps_ref[0] = jnp.dot(o, ws_ref[...], preferred_element_type=jnp.float32)
    pr_ref[0] = jnp.dot(o, wr_ref[...], preferred_element_type=jnp.float32)


def _node_final_body(xs_ref, agg_ref, wa_ref, wb_ref, w2_ref, w3_ref, b2_ref,
                     b3_ref, b1_ref, t2_ref, inv_ref, out_ref):
    h = (jnp.dot(xs_ref[0], wa_ref[...], preferred_element_type=jnp.float32)
         + jnp.dot(agg_ref[0, 0], wb_ref[...],
                   preferred_element_type=jnp.float32)
         + b1_ref[0])
    h = jnp.maximum(h, 0.0)
    h = jnp.dot(h, w2_ref[...], preferred_element_type=jnp.float32)
    h = jnp.maximum(h + b2_ref[...], 0.0)
    o = (jnp.dot(h, w3_ref[...], preferred_element_type=jnp.float32)
         + b3_ref[...] + t2_ref[0])
    out_ref[0] = o * inv_ref[0]


def _gather_body(ps, pr, snd3, rcv3, out,
                 idx_s, idx_r0, idx_r1,
                 ba0, bb0, ba1, bb1,
                 sa0, sb0, sa1, sb1, sw0, sw1):
    c = lax.axis_index("c")
    s = lax.axis_index("s")
    q0 = (c * _NS + s) * _NSUP
    base0 = q0 * _SUP
    slots = ((idx_r0, ba0, bb0, sa0, sb0, sw0),
             (idx_r1, ba1, bb1, sa1, sb1, sw1))

    pltpu.sync_copy(snd3.at[pl.ds(q0, _NSUP)], idx_s)

    def load_fire(u, slot, wait_write):
        idx_r, ba, bb, sa, sb, sw = slot
        pltpu.sync_copy(rcv3.at[q0 + u], idx_r)
        if wait_write:
            pltpu.make_async_copy(ba, out.at[pl.ds(0, _SUP)], sw).wait()
        for m in range(_MSUB):
            dst = pl.ds(m * _CH, _CH)
            pltpu.async_copy(ps.at[idx_s.at[u, m]], ba.at[dst], sa)
            pltpu.async_copy(pr.at[idx_r.at[m]], bb.at[dst], sb)

    def finish(u, slot):
        idx_r, ba, bb, sa, sb, sw = slot
        base = base0 + u * _SUP
        for m in range(_MSUB):
            dst = pl.ds(m * _CH, _CH)
            pltpu.make_async_copy(ps.at[idx_s.at[u, m]], ba.at[dst],
                                  sa).wait()
            pltpu.make_async_copy(pr.at[idx_r.at[m]], bb.at[dst],
                                  sb).wait()

        def addrow(i, carry2):
            for k in range(_H // 16):
                sl = pl.ds(k * 16, 16)
                ba[i, sl] = ba[i, sl] + bb[i, sl]
            return carry2
        lax.fori_loop(0, _SUP, addrow, 0)
        pltpu.async_copy(ba, out.at[pl.ds(base, _SUP)], sw)

    load_fire(0, slots[0], False)
    load_fire(1, slots[1], False)

    def body(t, carry):
        u0 = 2 * t
        finish(u0, slots[0])

        @pl.when(u0 + 2 < _NSUP)
        def _refire0():
            load_fire(u0 + 2, slots[0], True)

        finish(u0 + 1, slots[1])

        @pl.when(u0 + 3 < _NSUP)
        def _refire1():
            load_fire(u0 + 3, slots[1], True)
        return carry
    lax.fori_loop(0, _NSUP // 2, body, 0)
    if _NSUP % 2:
        finish(_NSUP - 1, slots[0])

    pltpu.make_async_copy(ba0, out.at[pl.ds(0, _SUP)], sw0).wait()
    pltpu.make_async_copy(ba1, out.at[pl.ds(0, _SUP)], sw1).wait()


def _scatter_body(en, rcv3l, zro, agg, sh, idx_a, dat0, dat1, ss0, ss1):
    c = lax.axis_index("c")
    s = lax.axis_index("s")

    @pl.when(s < 8)
    def _zero():
        pltpu.sync_copy(zro, sh.at[pl.ds(s * _ZROWS, _ZROWS)])

    @pl.when(s == 0)
    def _zero_tail():
        pltpu.sync_copy(zro.at[pl.ds(0, _ZTAIL)],
                        sh.at[pl.ds(8 * _ZROWS, _ZTAIL)])

    plsc.subcore_barrier()

    q0 = (c * _NS + s) * _NSUP
    base0 = q0 * _SUP
    slots = ((dat0, ss0), (dat1, ss1))

    pltpu.sync_copy(rcv3l.at[pl.ds(q0, _NSUP)], idx_a)

    def load_fire(u, slot):
        dat, sem = slot
        base = base0 + u * _SUP
        pltpu.sync_copy(en.at[pl.ds(base, _SUP)], dat)
        for m in range(_MSUB):
            src = pl.ds(m * _CH, _CH)
            pltpu.async_copy(dat.at[src], sh.at[idx_a.at[u, m]], sem,
                             add=True)

    def wait_sc(slot):
        dat, sem = slot
        for m in range(_MSUB):
            src = pl.ds(m * _CH, _CH)
            pltpu.make_async_copy(dat.at[src], sh.at[idx_a.at[0, m]],
                                  sem).wait()

    load_fire(0, slots[0])

    def body(t, carry):
        u0 = 2 * t
        load_fire(u0 + 1, slots[1])
        wait_sc(slots[0])

        @pl.when(u0 + 2 < _NSUP)
        def _refire0():
            load_fire(u0 + 2, slots[0])
        wait_sc(slots[1])
        return carry
    lax.fori_loop(0, _NSUP // 2, body, 0)
    if _NSUP % 2:
        wait_sc(slots[0])

    plsc.subcore_barrier()

    @pl.when(s < 8)
    def _writeout():
        rows = pl.ds(s * _ZROWS, _ZROWS)
        pltpu.sync_copy(sh.at[rows], agg.at[c, rows])

    @pl.when(s == 0)
    def _write_tail():
        tail = pl.ds(8 * _ZROWS, _ZTAIL)
        pltpu.sync_copy(sh.at[tail], agg.at[c, tail])


@functools.cache
def _sc_kernels():
    mesh = plsc.VectorSubcoreMesh(core_axis_name="c", subcore_axis_name="s",
                                  num_cores=_NC, num_subcores=_NS)
    gather = pl.kernel(
        _gather_body,
        mesh=mesh,
        out_type=jax.ShapeDtypeStruct((_EH, _H), jnp.float32),
        scratch_types=[
            pltpu.VMEM((_NSUP, _MSUB, _CH), jnp.int32),
            pltpu.VMEM((_MSUB, _CH), jnp.int32),
            pltpu.VMEM((_MSUB, _CH), jnp.int32),
            pltpu.VMEM((_SUP, _H), jnp.float32),
            pltpu.VMEM((_SUP, _H), jnp.float32),
            pltpu.VMEM((_SUP, _H), jnp.float32),
            pltpu.VMEM((_SUP, _H), jnp.float32),
            pltpu.SemaphoreType.DMA,
            pltpu.SemaphoreType.DMA,
            pltpu.SemaphoreType.DMA,
            pltpu.SemaphoreType.DMA,
            pltpu.SemaphoreType.DMA,
            pltpu.SemaphoreType.DMA,
        ],
    )
    scatter = pl.kernel(
        _scatter_body,
        mesh=mesh,
        out_type=jax.ShapeDtypeStruct((2, _APAD, _H), jnp.float32),
        scratch_types=[
            pltpu.VMEM_SHARED((_APAD, _H), jnp.float32),
            pltpu.VMEM((_NSUP, _MSUB, _CH), jnp.int32),
            pltpu.VMEM((_SUP, _H), jnp.float32),
            pltpu.VMEM((_SUP, _H), jnp.float32),
            pltpu.SemaphoreType.DMA,
            pltpu.SemaphoreType.DMA,
        ],
    )
    return gather, scatter


_f32 = jnp.float32


def _proj_call(xs, ws, wr):
    nb = _NH // _BP
    return pl.pallas_call(
        _proj_body,
        grid=(nb,),
        in_specs=[
            pl.BlockSpec((_BP, _H), lambda i: (i, 0)),
            pl.BlockSpec((_H, _H), lambda i: (0, 0)),
            pl.BlockSpec((_H, _H), lambda i: (0, 0)),
        ],
        out_specs=[
            pl.BlockSpec((_BP, _H), lambda i: (i, 0)),
            pl.BlockSpec((_BP, _H), lambda i: (i, 0)),
        ],
        out_shape=[
            jax.ShapeDtypeStruct((_NH, _H), _f32),
            jax.ShapeDtypeStruct((_NH, _H), _f32),
        ],
    )(xs, ws, wr)


def _edge_call(es, g, w1, w2, w3, b2, b3, eb1):
    nb = _EH // _BE
    bpg = _EPG // _BE
    return pl.pallas_call(
        _edge_body,
        grid=(nb,),
        in_specs=[
            pl.BlockSpec((_BE, _H), lambda i: (i, 0)),
            pl.BlockSpec((_BE, _H), lambda i: (i, 0)),
            pl.BlockSpec((_H, _H), lambda i: (0, 0)),
            pl.BlockSpec((_H, _H), lambda i: (0, 0)),
            pl.BlockSpec((_H, _H), lambda i: (0, 0)),
            pl.BlockSpec((1, _H), lambda i: (0, 0)),
            pl.BlockSpec((1, _H), lambda i: (0, 0)),
            pl.BlockSpec((1, 1, _H), lambda i: (i // bpg, 0, 0)),
        ],
        out_specs=pl.BlockSpec((_BE, _H), lambda i: (i, 0)),
        out_shape=jax.ShapeDtypeStruct((_EH, _H), _f32),
    )(es, g, w1, w2, w3, b2, b3, eb1)


def _node_call(xs3, agg3, wa, wb, w2, w3, b2, b3, nb1, ws, wr):
    return pl.pallas_call(
        _node_body,
        grid=(_B // 2,),
        in_specs=[
            pl.BlockSpec((1, _NPG, _H), lambda b: (b, 0, 0)),
            pl.BlockSpec((1, 1, _NPG, _H), lambda b: (b // 2, b % 2, 0, 0)),
            pl.BlockSpec((_H, _H), lambda b: (0, 0)),
            pl.BlockSpec((_H, _H), lambda b: (0, 0)),
            pl.BlockSpec((_H, _H), lambda b: (0, 0)),
            pl.BlockSpec((_H, _H), lambda b: (0, 0)),
            pl.BlockSpec((1, _H), lambda b: (0, 0)),
            pl.BlockSpec((1, _H), lambda b: (0, 0)),
            pl.BlockSpec((1, 1, _H), lambda b: (b, 0, 0)),
            pl.BlockSpec((_H, _H), lambda b: (0, 0)),
            pl.BlockSpec((_H, _H), lambda b: (0, 0)),
        ],
        out_specs=[
            pl.BlockSpec((1, _NPG, _H), lambda b: (b, 0, 0)),
            pl.BlockSpec((1, _NPG, _H), lambda b: (b, 0, 0)),
            pl.BlockSpec((1, _NPG, _H), lambda b: (b, 0, 0)),
        ],
        out_shape=[
            jax.ShapeDtypeStruct((_B // 2, _NPG, _H), _f32),
            jax.ShapeDtypeStruct((_B // 2, _NPG, _H), _f32),
            jax.ShapeDtypeStruct((_B // 2, _NPG, _H), _f32),
        ],
    )(xs3, agg3, wa, wb, w2, w3, b2, b3, nb1, ws, wr)


def _node_final_call(xs3, agg3, wa, wb, w2, w3, b2, b3, nb1, t2, inv):
    return pl.pallas_call(
        _node_final_body,
        grid=(_B // 2,),
        in_specs=[
            pl.BlockSpec((1, _NPG, _H), lambda b: (b, 0, 0)),
            pl.BlockSpec((1, 1, _NPG, _H), lambda b: (b // 2, b % 2, 0, 0)),
            pl.BlockSpec((_H, _H), lambda b: (0, 0)),
            pl.BlockSpec((_H, _H), lambda b: (0, 0)),
            pl.BlockSpec((_H, _H), lambda b: (0, 0)),
            pl.BlockSpec((_H, _H), lambda b: (0, 0)),
            pl.BlockSpec((1, _H), lambda b: (0, 0)),
            pl.BlockSpec((1, _H), lambda b: (0, 0)),
            pl.BlockSpec((1, 1, _H), lambda b: (b, 0, 0)),
            pl.BlockSpec((1, 1, _H), lambda b: (b, 0, 0)),
            pl.BlockSpec((1, 1, _H), lambda b: (b, 0, 0)),
        ],
        out_specs=pl.BlockSpec((1, _NPG, _H), lambda b: (b, 0, 0)),
        out_shape=jax.ShapeDtypeStruct((_B // 2, _NPG, _H), _f32),
    )(xs3, agg3, wa, wb, w2, w3, b2, b3, nb1, t2, inv)


def _temb_call(t2d, gfp, twa, twb, tb, wsum, wn1, be, bn):
    return pl.pallas_call(
        _temb_body,
        out_shape=[
            jax.ShapeDtypeStruct((3, _B, _H), _f32),
            jax.ShapeDtypeStruct((3, _B, _H), _f32),
            jax.ShapeDtypeStruct((_B, _H), _f32),
            jax.ShapeDtypeStruct((_B, _H), _f32),
        ],
    )(t2d, gfp, twa, twb, tb, wsum, wn1, be, bn)


def kernel(x, edges, t, senders, receivers, params):
    lp = [params['layer%d' % l] for l in range(3)]
    w1e = [p['e_W'][0][0:_H] for p in lp]
    w1s = [p['e_W'][0][_H:2 * _H] for p in lp]
    w1r = [p['e_W'][0][2 * _H:3 * _H] for p in lp]
    wsum = jnp.stack([w1e[l] + w1s[l] + w1r[l] for l in range(3)])
    wn1a = [p['n_W'][0][0:_H] for p in lp]
    wn1b = [p['n_W'][0][_H:2 * _H] for p in lp]
    wn1 = jnp.stack(wn1a)
    gfp = jnp.stack([p['gfp_W'] for p in lp]).reshape(3, 1, _H // 2)
    twa = jnp.stack([p['t_W'][0:_H // 2] for p in lp])
    twb = jnp.stack([p['t_W'][_H // 2:] for p in lp])
    tb = jnp.stack([p['t_b'] for p in lp]).reshape(3, 1, _H)
    be = jnp.stack([p['e_b'][0] for p in lp]).reshape(3, 1, _H)
    bn = jnp.stack([p['n_b'][0] for p in lp]).reshape(3, 1, _H)
    t2d = t.reshape(_B, 1)

    eb1, nb1, temb2, inv = _temb_call(t2d, gfp, twa, twb, tb, wsum, wn1,
                                      be, bn)
    eb1_3 = [eb1[l].reshape(_B, 1, _H) for l in range(3)]
    nb1_3 = [nb1[l].reshape(_B, 1, _H) for l in range(3)]
    temb2_3 = temb2.reshape(_B, 1, _H)
    inv_3 = inv.reshape(_B, 1, _H)

    zro = jnp.zeros((_ZROWS, _H), _f32)
    snd_l = (senders.astype(jnp.int32) % _NH).reshape(-1, _MSUB, _CH)
    rcv_g = receivers.astype(jnp.int32)
    rcv_l = (rcv_g % _NH).reshape(-1, _MSUB, _CH)
    rcv_c = (rcv_g % _NPC).reshape(-1, _MSUB, _CH)
    nsup_h = _EH // _SUP
    snd3 = [snd_l[h * nsup_h:(h + 1) * nsup_h] for h in range(2)]
    rcv3 = [rcv_l[h * nsup_h:(h + 1) * nsup_h] for h in range(2)]
    rcv3c = [rcv_c[h * nsup_h:(h + 1) * nsup_h] for h in range(2)]

    gather_k, scatter_k = _sc_kernels()

    xs = [x[:_NH], x[_NH:]]
    es = [edges[:_EH], edges[_EH:]]
    ps = [None, None]
    pr = [None, None]
    for h in range(2):
        ps[h], pr[h] = _proj_call(xs[h], w1s[0], w1r[0])
    out3 = [None, None]
    for l in range(3):
        p = lp[l]
        eb2 = p['e_b'][1].reshape(1, _H)
        eb3 = p['e_b'][2].reshape(1, _H)
        nb2 = p['n_b'][1].reshape(1, _H)
        nb3 = p['n_b'][2].reshape(1, _H)
        for h in range(2):
            g = gather_k(ps[h], pr[h], snd3[h], rcv3[h])
            en = _edge_call(es[h], g, w1e[l], p['e_W'][1], p['e_W'][2],
                            eb2, eb3, eb1_3[l][4 * h:4 * h + 4])
            agg = scatter_k(en, rcv3c[h], zro).reshape(2, 2, _NPG, _H)
            xs3 = xs[h].reshape(_B // 2, _NPG, _H)
            if l < 2:
                xn3, ps3, pr3 = _node_call(
                    xs3, agg, wn1a[l], wn1b[l], p['n_W'][1], p['n_W'][2],
                    nb2, nb3, nb1_3[l][4 * h:4 * h + 4],
                    w1s[l + 1], w1r[l + 1])
                xs[h] = xn3.reshape(_NH, _H)
                ps[h] = ps3.reshape(_NH, _H)
                pr[h] = pr3.reshape(_NH, _H)
            else:
                out3[h] = _node_final_call(
                    xs3, agg, wn1a[l], wn1b[l], p['n_W'][1], p['n_W'][2],
                    nb2, nb3, nb1_3[l][4 * h:4 * h + 4],
                    temb2_3[4 * h:4 * h + 4], inv_3[4 * h:4 * h + 4])
            es[h] = en
    return jnp.concatenate([out3[0].reshape(_B // 2, _NPG * _H),
                            out3[1].reshape(_B // 2, _NPG * _H)], axis=0)
